# baseline (speedup 1.0000x reference)
import functools

import jax
import jax.numpy as jnp
from jax import lax
from jax.experimental import pallas as pl
from jax.experimental.pallas import tpu as pltpu
from jax.experimental.pallas import tpu_sc as plsc

_NC, _NS = 2, 16
_NW = _NC * _NS
_CB = 128


def _mesh():
  return plsc.VectorSubcoreMesh(core_axis_name="c", subcore_axis_name="s",
                                num_cores=_NC, num_subcores=_NS)


_NB = 2


@functools.cache
def _segsum(n_rows: int, feat: int, e_pad: int):
  cb = _CB
  rpw = -(-(n_rows + 1) // (_NS * 8)) * 8
  npad = rpw * _NS
  cpw = e_pad // (_NW * cb)
  assert cpw % _NB == 0 and cpw > _NB
  sz_last = n_rows - (_NS - 1) * rpw

  @functools.partial(
      pl.kernel, mesh=_mesh(),
      out_type=jax.ShapeDtypeStruct((_NC, n_rows, feat), jnp.float32),
      scratch_types=[
          pltpu.VMEM((cpw, cb), jnp.int32),
          pltpu.VMEM((cpw, cb), jnp.int32),
          [pltpu.VMEM((cb, feat), jnp.float32)] * _NB,
          pltpu.VMEM_SHARED((npad, feat), jnp.float32),
          [pltpu.SemaphoreType.DMA] * _NB,
          [pltpu.SemaphoreType.DMA] * _NB,
      ],
  )
  def seg(u_hbm, gidx_hbm, sidx_hbm, zeros_hbm, out_hbm,
          gv, sv, rows_v, acc, gsem, ssem):
    c = lax.axis_index("c")
    s = lax.axis_index("s")
    w = c * _NS + s
    pltpu.sync_copy(zeros_hbm, acc.at[pl.ds(s * rpw, rpw)])
    cr0 = pl.multiple_of(w * cpw, 8)
    pltpu.sync_copy(gidx_hbm.at[pl.ds(cr0, cpw)], gv)
    pltpu.sync_copy(sidx_hbm.at[pl.ds(cr0, cpw)], sv)
    plsc.subcore_barrier()

    pltpu.async_copy(u_hbm.at[gv.at[0]], rows_v[0], gsem[0])

    def block(blk, carry):
      j0 = blk * _NB
      for b in range(_NB):
        j = j0 + b
        o = 1 - b
        pltpu.make_async_copy(u_hbm.at[gv.at[b]], rows_v[b], gsem[b]).wait()
        pltpu.async_copy(rows_v[b], acc.at[sv.at[j]], ssem[b], add=True)

        @pl.when(j >= 1)
        def _():
          pltpu.make_async_copy(rows_v[o], acc.at[sv.at[j]], ssem[o]).wait()

        @pl.when(j + 1 < cpw)
        def _():
          pltpu.async_copy(u_hbm.at[gv.at[j + 1]], rows_v[o], gsem[o])

      return carry

    lax.fori_loop(0, cpw // _NB, block, 0)
    pltpu.make_async_copy(rows_v[1], acc.at[sv.at[0]], ssem[1]).wait()
    plsc.subcore_barrier()

    @pl.when(s < _NS - 1)
    def _():
      r0 = s * rpw
      pltpu.sync_copy(acc.at[pl.ds(r0, rpw)], out_hbm.at[c].at[pl.ds(r0, rpw)])

    @pl.when(s == _NS - 1)
    def _():
      r0 = (_NS - 1) * rpw
      pltpu.sync_copy(acc.at[pl.ds(r0, sz_last)],
                      out_hbm.at[c].at[pl.ds(r0, sz_last)])

  return seg


@functools.cache
def _degree(n_rows: int, feat: int, e_pad: int):
  cb = _CB
  rpw = -(-(n_rows + 1) // (_NS * 8)) * 8
  npad = rpw * _NS
  cpw = e_pad // (_NW * cb)
  sz_last = n_rows - (_NS - 1) * rpw

  @functools.partial(
      pl.kernel, mesh=_mesh(),
      out_type=jax.ShapeDtypeStruct((_NC, n_rows, feat), jnp.float32),
      scratch_types=[
          pltpu.VMEM((cpw, cb), jnp.int32),
          pltpu.VMEM((cb, feat), jnp.float32),
          pltpu.VMEM_SHARED((npad, feat), jnp.float32),
      ],
  )
  def deg(sidx_hbm, ones_hbm, zeros_hbm, out_hbm, sv, ones_v, acc):
    c = lax.axis_index("c")
    s = lax.axis_index("s")
    w = c * _NS + s
    pltpu.sync_copy(ones_hbm, ones_v)
    pltpu.sync_copy(zeros_hbm, acc.at[pl.ds(s * rpw, rpw)])
    cr0 = pl.multiple_of(w * cpw, 8)
    pltpu.sync_copy(sidx_hbm.at[pl.ds(cr0, cpw)], sv)
    plsc.subcore_barrier()

    def chunk(j, carry):
      pltpu.sync_copy(ones_v, acc.at[sv.at[j]], add=True)
      return carry

    lax.fori_loop(0, cpw, chunk, 0)
    plsc.subcore_barrier()

    @pl.when(s < _NS - 1)
    def _():
      r0 = s * rpw
      pltpu.sync_copy(acc.at[pl.ds(r0, rpw)], out_hbm.at[c].at[pl.ds(r0, rpw)])

    @pl.when(s == _NS - 1)
    def _():
      r0 = (_NS - 1) * rpw
      pltpu.sync_copy(acc.at[pl.ds(r0, sz_last)],
                      out_hbm.at[c].at[pl.ds(r0, sz_last)])

  return deg


@functools.cache
def _gather2(n_rows: int, feat: int, e_pad: int):
  cb = _CB
  cpw = e_pad // (_NW * cb)

  @functools.partial(
      pl.kernel, mesh=_mesh(),
      out_type=(jax.ShapeDtypeStruct((e_pad, feat), jnp.float32),
                jax.ShapeDtypeStruct((e_pad, feat), jnp.float32)),
      scratch_types=[
          pltpu.VMEM((cpw, cb), jnp.int32),
          pltpu.VMEM((cpw, cb), jnp.int32),
          [pltpu.VMEM((cb, feat), jnp.float32)] * 2,
          [pltpu.VMEM((cb, feat), jnp.float32)] * 2,
          [pltpu.SemaphoreType.DMA] * 2,
          [pltpu.SemaphoreType.DMA] * 2,
          [pltpu.SemaphoreType.DMA] * 2,
          [pltpu.SemaphoreType.DMA] * 2,
      ],
  )
  def gat(tab_hbm, aidx_hbm, bidx_hbm, oa_hbm, ob_hbm,
          av, bv, rows_a, rows_b, gsa, gsb, ssa, ssb):
    c = lax.axis_index("c")
    s = lax.axis_index("s")
    w = c * _NS + s
    w_base = pl.multiple_of(w * (cpw * cb), 8)
    cr0 = pl.multiple_of(w * cpw, 8)
    pltpu.sync_copy(aidx_hbm.at[pl.ds(cr0, cpw)], av)
    pltpu.sync_copy(bidx_hbm.at[pl.ds(cr0, cpw)], bv)
    pltpu.async_copy(tab_hbm.at[av.at[0]], rows_a[0], gsa[0])
    pltpu.async_copy(tab_hbm.at[bv.at[0]], rows_b[0], gsb[0])

    def block(blk, carry):
      j0 = blk * 2
      for b in range(2):
        j = j0 + b
        o = 1 - b
        base = pl.multiple_of(w_base + j * cb, 8)
        obase = pl.multiple_of(w_base + (j - 1) * cb, 8)
        pltpu.make_async_copy(tab_hbm.at[av.at[b]], rows_a[b], gsa[b]).wait()
        pltpu.async_copy(rows_a[b], oa_hbm.at[pl.ds(base, cb)], ssa[b])
        pltpu.make_async_copy(tab_hbm.at[bv.at[b]], rows_b[b], gsb[b]).wait()
        pltpu.async_copy(rows_b[b], ob_hbm.at[pl.ds(base, cb)], ssb[b])

        @pl.when(j >= 1)
        def _():
          pltpu.make_async_copy(rows_a[o], oa_hbm.at[pl.ds(obase, cb)],
                                ssa[o]).wait()
          pltpu.make_async_copy(rows_b[o], ob_hbm.at[pl.ds(obase, cb)],
                                ssb[o]).wait()

        @pl.when(j + 1 < cpw)
        def _():
          pltpu.async_copy(tab_hbm.at[av.at[j + 1]], rows_a[o], gsa[o])
          pltpu.async_copy(tab_hbm.at[bv.at[j + 1]], rows_b[o], gsb[o])

      return carry

    lax.fori_loop(0, cpw // 2, block, 0)
    pltpu.make_async_copy(rows_a[1], oa_hbm.at[pl.ds(w_base, cb)],
                          ssa[1]).wait()
    pltpu.make_async_copy(rows_b[1], ob_hbm.at[pl.ds(w_base, cb)],
                          ssb[1]).wait()

  return gat


def _vspec(bn, *trail):
  return pl.BlockSpec((bn,) + trail, lambda i: (i,) + (0,) * len(trail))


def _wspec(shape):
  return pl.BlockSpec(shape, lambda i: (0,) * len(shape))


def _k_prescale(degp, x_seq, bn):
  t, n, f = x_seq.shape
  fd = degp.shape[2]

  def body(dp_ref, x_ref, dv_ref, xs_ref):
    deg = dp_ref[0, :, 0] + dp_ref[1, :, 0]
    dv = jnp.where(deg > 0, 1.0 / jnp.sqrt(jnp.maximum(deg, 1e-12)), 0.0)
    dv_ref[...] = dv[:, None]
    xs_ref[...] = x_ref[...] * dv[None, :, None]

  return pl.pallas_call(
      body,
      grid=(n // bn,),
      in_specs=[pl.BlockSpec((2, bn, fd), lambda i: (0, i, 0)),
                pl.BlockSpec((t, bn, f), lambda i: (0, i, 0))],
      out_specs=[_vspec(bn, 1), pl.BlockSpec((t, bn, f), lambda i: (0, i, 0))],
      out_shape=[jax.ShapeDtypeStruct((n, 1), jnp.float32),
                 jax.ShapeDtypeStruct((t, n, f), jnp.float32)],
  )(degp, x_seq)


def _k_mid(p1, dinv, bn):
  _, n, f = p1.shape

  def body(p_ref, dv_ref, lap_ref, m1_ref):
    ps = p_ref[0] + p_ref[1]
    dv = dv_ref[...]
    lap_ref[...] = -dv * ps
    m1_ref[...] = (dv * dv) * ps

  return pl.pallas_call(
      body,
      grid=(n // bn,),
      in_specs=[pl.BlockSpec((2, bn, f), lambda i: (0, i, 0)), _vspec(bn, 1)],
      out_specs=[_vspec(bn, f), _vspec(bn, f)],
      out_shape=[jax.ShapeDtypeStruct((n, f), jnp.float32)] * 2,
  )(p1, dinv)


def _k_cheb(v, lapv, p2, dinv, a0m2, a1, a2x2, bias, bn, t_idx):
  _, n, f = v.shape
  v_spec = pl.BlockSpec((1, bn, f), lambda i, t=t_idx: (t, i, 0))
  fo = a1.shape[1]

  def body(v_ref, lap_ref, p2_ref, dv_ref, a0_ref, a1_ref, a2_ref, b_ref,
           o_ref):
    vv = v_ref[...].reshape(bn, f)
    dv = dv_ref[...]
    l2 = dv * (p2_ref[0] + p2_ref[1])
    acc = jnp.dot(vv, a0_ref[...], preferred_element_type=jnp.float32)
    acc += jnp.dot(lap_ref[...], a1_ref[...], preferred_element_type=jnp.float32)
    acc += jnp.dot(l2, a2_ref[...], preferred_element_type=jnp.float32)
    o_ref[...] = acc + b_ref[...]

  return pl.pallas_call(
      body,
      grid=(n // bn,),
      in_specs=[v_spec, _vspec(bn, f),
                pl.BlockSpec((2, bn, f), lambda i: (0, i, 0)), _vspec(bn, 1),
                _wspec(a0m2.shape), _wspec(a1.shape), _wspec(a2x2.shape),
                _wspec(bias.shape)],
      out_specs=_vspec(bn, fo),
      out_shape=jax.ShapeDtypeStruct((n, fo), jnp.float32),
  )(v, lapv, p2, dinv, a0m2, a1, a2x2, bias)


def _k_t0(cx0, dinv, b_hz, b_hn, hg, bn):
  n = cx0.shape[0]

  def body(cx_ref, dv_ref, bz_ref, bnn_ref, h_ref, u_ref):
    cx = cx_ref[...]
    z = jax.nn.sigmoid(cx[:, :hg] + bz_ref[...])
    htil = jnp.tanh(cx[:, 2 * hg:] + bnn_ref[...])
    h1 = (1.0 - z) * htil
    zer = jnp.zeros((h1.shape[0], 2 * hg - hg), jnp.float32)
    h_ref[...] = jnp.concatenate([h1, zer], axis=1)
    u_ref[...] = jnp.concatenate([dv_ref[...] * h1, zer], axis=1)

  return pl.pallas_call(
      body,
      grid=(n // bn,),
      in_specs=[_vspec(bn, 3 * hg), _vspec(bn, 1), _wspec(b_hz.shape),
                _wspec(b_hn.shape)],
      out_specs=[_vspec(bn, 2 * hg), _vspec(bn, 2 * hg)],
      out_shape=[jax.ShapeDtypeStruct((n, 2 * hg), jnp.float32)] * 2,
  )(cx0, dinv, b_hz, b_hn)


def _k_zr(cx, h, laph, p2h, dinv, b0m2, b1w, b2x2, bzr, hg, bn):
  n = cx.shape[0]
  hp = h.shape[1]

  def body(cx_ref, h_ref, lap_ref, p2_ref, dv_ref, b0_ref, b1_ref, b2_ref,
           bb_ref, z_ref, hr_ref, u_ref):
    dv = dv_ref[...]
    hh = h_ref[...][:, :hg]
    l2 = dv * (p2_ref[0, :, :hg] + p2_ref[1, :, :hg])
    ch = jnp.dot(hh, b0_ref[...], preferred_element_type=jnp.float32)
    ch += jnp.dot(lap_ref[...][:, :hg], b1_ref[...],
                  preferred_element_type=jnp.float32)
    ch += jnp.dot(l2, b2_ref[...], preferred_element_type=jnp.float32)
    ch += bb_ref[...]
    cxv = cx_ref[...]
    z = jax.nn.sigmoid(cxv[:, :hg] + ch[:, :hg])
    r = jax.nn.sigmoid(cxv[:, hg:2 * hg] + ch[:, hg:])
    hr = hh * r
    z_ref[...] = z
    hr_ref[...] = hr
    u_ref[...] = jnp.concatenate(
        [dv * hr, jnp.zeros((hr.shape[0], hp - hg), jnp.float32)], axis=1)

  return pl.pallas_call(
      body,
      grid=(n // bn,),
      in_specs=[_vspec(bn, 3 * hg), _vspec(bn, hp), _vspec(bn, hp),
                pl.BlockSpec((2, bn, hp), lambda i: (0, i, 0)), _vspec(bn, 1),
                _wspec(b0m2.shape), _wspec(b1w.shape), _wspec(b2x2.shape),
                _wspec(bzr.shape)],
      out_specs=[_vspec(bn, hg), _vspec(bn, hg), _vspec(bn, hp)],
      out_shape=[jax.ShapeDtypeStruct((n, hg), jnp.float32),
                 jax.ShapeDtypeStruct((n, hg), jnp.float32),
                 jax.ShapeDtypeStruct((n, hp), jnp.float32)],
  )(cx, h, laph, p2h, dinv, b0m2, b1w, b2x2, bzr)


def _k_upd(cx, h, z, hr, lapn, p2n, dinv, c0m2, c1w, c2x2, bnn, hg, bn):
  n = cx.shape[0]
  hp = h.shape[1]

  def body(cx_ref, h_ref, z_ref, hr_ref, lap_ref, p2_ref, dv_ref, c0_ref,
           c1_ref, c2_ref, bb_ref, hn_ref, u_ref):
    dv = dv_ref[...]
    l2 = dv * (p2_ref[0, :, :hg] + p2_ref[1, :, :hg])
    ch = jnp.dot(hr_ref[...], c0_ref[...], preferred_element_type=jnp.float32)
    ch += jnp.dot(lap_ref[...][:, :hg], c1_ref[...],
                  preferred_element_type=jnp.float32)
    ch += jnp.dot(l2, c2_ref[...], preferred_element_type=jnp.float32)
    htil = jnp.tanh(cx_ref[...][:, 2 * hg:] + ch + bb_ref[...])
    z = z_ref[...]
    hn = z * h_ref[...][:, :hg] + (1.0 - z) * htil
    zer = jnp.zeros((hn.shape[0], hp - hg), jnp.float32)
    hn_ref[...] = jnp.concatenate([hn, zer], axis=1)
    u_ref[...] = jnp.concatenate([dv * hn, zer], axis=1)

  return pl.pallas_call(
      body,
      grid=(n // bn,),
      in_specs=[_vspec(bn, 3 * hg), _vspec(bn, hp), _vspec(bn, hg),
                _vspec(bn, hg), _vspec(bn, hp),
                pl.BlockSpec((2, bn, hp), lambda i: (0, i, 0)), _vspec(bn, 1),
                _wspec(c0m2.shape), _wspec(c1w.shape), _wspec(c2x2.shape),
                _wspec(bnn.shape)],
      out_specs=[_vspec(bn, hp), _vspec(bn, hp)],
      out_shape=[jax.ShapeDtypeStruct((n, hp), jnp.float32)] * 2,
  )(cx, h, z, hr, lapn, p2n, dinv, c0m2, c1w, c2x2, bnn)


def _k_decoder(hs, hd, tc16, st, w_s, w_d, w_tblk, w_st, b_ih, w_hh, b_hh,
               w1, b1, w2, b2, e, t_steps, hg, hdec, be):
  hp = hs.shape[1]
  fs = w_st.shape[0]
  ftt = tc16.shape[1]

  def body(hs_ref, hd_ref, tc_ref, st_ref, ws_ref, wd_ref, wt_ref, wst_ref,
           bih_ref, whh_ref, bhh_ref, w1_ref, b1_ref, w2_ref, b2_ref, o_ref):
    gi_base = jnp.dot(hs_ref[...][:, :hg], ws_ref[...],
                      preferred_element_type=jnp.float32)
    gi_base += jnp.dot(hd_ref[...][:, :hg], wd_ref[...],
                       preferred_element_type=jnp.float32)
    gi_base += jnp.dot(st_ref[...], wst_ref[...],
                       preferred_element_type=jnp.float32)
    gi_base += bih_ref[...]
    gt_all = jnp.dot(tc_ref[...], wt_ref[...],
                     preferred_element_type=jnp.float32)
    h = jnp.zeros((be, hdec), jnp.float32)
    cols = []
    for t in range(t_steps):
      gi = gi_base + gt_all[:, 3 * hdec * t:3 * hdec * (t + 1)]
      gh = jnp.dot(h, whh_ref[...], preferred_element_type=jnp.float32)
      gh += bhh_ref[...]
      r = jax.nn.sigmoid(gi[:, :hdec] + gh[:, :hdec])
      z = jax.nn.sigmoid(gi[:, hdec:2 * hdec] + gh[:, hdec:2 * hdec])
      nn = jnp.tanh(gi[:, 2 * hdec:] + r * gh[:, 2 * hdec:])
      h = (1.0 - z) * nn + z * h
      hid = jax.nn.relu(
          jnp.dot(h, w1_ref[...], preferred_element_type=jnp.float32)
          + b1_ref[...])
      cols.append(jnp.dot(hid, w2_ref[...], preferred_element_type=jnp.float32)
                  + b2_ref[...])
    o_ref[...] = jnp.concatenate(cols, axis=1)

  return pl.pallas_call(
      body,
      grid=(e // be,),
      in_specs=[_vspec(be, hp), _vspec(be, hp), _vspec(be, ftt),
                _vspec(be, fs),
                _wspec(w_s.shape), _wspec(w_d.shape), _wspec(w_tblk.shape),
                _wspec(w_st.shape), _wspec(b_ih.shape), _wspec(w_hh.shape),
                _wspec(b_hh.shape), _wspec(w1.shape), _wspec(b1.shape),
                _wspec(w2.shape), _wspec(b2.shape)],
      out_specs=_vspec(be, t_steps),
      out_shape=jax.ShapeDtypeStruct((e, t_steps), jnp.float32),
  )(hs, hd, tc16, st, w_s, w_d, w_tblk, w_st, b_ih, w_hh, b_hh, w1, b1, w2, b2)


def kernel(x_seq, edge_index, time_seq, static_feats,
           W_xz, b_xz, W_hz, b_hz, W_xr, b_xr, W_hr, b_hr,
           W_xh, b_xh, W_hn, b_hn,
           W_ih, W_hh, b_ih, b_hh, W1, b1, W2, b2):
  t_steps, n, f_node = x_seq.shape
  e = edge_index.shape[1]
  hg = W_hz.shape[2]
  hp = 2 * hg
  hdec = W_hh.shape[0] // 3
  ft = time_seq.shape[2]

  cpw = -(-e // (_NW * _CB))
  e_pad = _NW * cpw * _CB
  pad = e_pad - e
  bn = 2000
  be = 2000

  src = edge_index[0]
  dst = edge_index[1]
  zpad_i = jnp.zeros((pad,), jnp.int32)
  npad_i = jnp.full((pad,), n, jnp.int32)
  src_g = jnp.concatenate([src, zpad_i]).reshape(-1, _CB)
  dst_g = jnp.concatenate([dst, zpad_i]).reshape(-1, _CB)
  src_s = jnp.concatenate([src, npad_i]).reshape(-1, _CB)
  dst_s = jnp.concatenate([dst, npad_i]).reshape(-1, _CB)

  rpw = -(-(n + 1) // (_NS * 8)) * 8
  z_fn = jnp.zeros((rpw, f_node), jnp.float32)
  ones_row = jnp.ones((_CB, f_node), jnp.float32)

  seg_fn = _segsum(n, f_node, e_pad)
  deg_k = _degree(n, f_node, e_pad)
  gat_k = _gather2(n, f_node, e_pad)

  degp = deg_k(src_s, ones_row, z_fn)
  dinv, xs = _k_prescale(degp, x_seq, bn)

  a_k = [jnp.concatenate([W_xz[k], W_xr[k], W_xh[k]], axis=1) for k in range(3)]
  a0m2, a1w, a2x2 = a_k[0] - a_k[2], a_k[1], 2.0 * a_k[2]
  bx = jnp.concatenate([b_xz, b_xr, b_xh]).reshape(1, 3 * hg)
  b_k = [jnp.concatenate([W_hz[k], W_hr[k]], axis=1) for k in range(3)]
  b0m2, b1w, b2x2 = b_k[0] - b_k[2], b_k[1], 2.0 * b_k[2]
  bzr = jnp.concatenate([b_hz, b_hr]).reshape(1, 2 * hg)
  c0m2, c1w, c2x2 = W_hn[0] - W_hn[2], W_hn[1], 2.0 * W_hn[2]
  bz2 = b_hz.reshape(1, hg)
  bn2 = b_hn.reshape(1, hg)

  cxs = []
  for t in range(t_steps):
    p1 = seg_fn(xs[t], src_g, dst_s, z_fn)
    lapx, m1x = _k_mid(p1, dinv, bn)
    p2 = seg_fn(m1x, src_g, dst_s, z_fn)
    cxs.append(_k_cheb(x_seq, lapx, p2, dinv, a0m2, a1w, a2x2, bx, bn, t))

  h_cur, u_cur = _k_t0(cxs[0], dinv, bz2, bn2, hg, bn)
  for t in range(1, t_steps):
    p1h = seg_fn(u_cur, src_g, dst_s, z_fn)
    laph, m1h = _k_mid(p1h, dinv, bn)
    p2h = seg_fn(m1h, src_g, dst_s, z_fn)
    z_gate, hr, uhr = _k_zr(cxs[t], h_cur, laph, p2h, dinv, b0m2, b1w, b2x2,
                            bzr, hg, bn)
    p1n = seg_fn(uhr, src_g, dst_s, z_fn)
    lapn, m1n = _k_mid(p1n, dinv, bn)
    p2n = seg_fn(m1n, src_g, dst_s, z_fn)
    h_cur, u_cur = _k_upd(cxs[t], h_cur, z_gate, hr, lapn, p2n, dinv, c0m2,
                          c1w, c2x2, bn2, hg, bn)

  hs, hd = gat_k(h_cur, src_g, dst_g)
  tc16 = jnp.transpose(time_seq, (1, 0, 2)).reshape(e, t_steps * ft)
  w_s = W_ih[:, :hg].T
  w_d = W_ih[:, hg:2 * hg].T
  w_t = W_ih[:, 2 * hg:2 * hg + ft].T
  w_st = W_ih[:, 2 * hg + ft:].T
  w_tblk = jnp.zeros((t_steps * ft, t_steps * 3 * hdec), jnp.float32)
  for t in range(t_steps):
    w_tblk = w_tblk.at[ft * t:ft * (t + 1),
                       3 * hdec * t:3 * hdec * (t + 1)].set(w_t)
  out_et = _k_decoder(hs, hd, tc16, static_feats,
                      w_s, w_d, w_tblk, w_st, b_ih.reshape(1, -1),
                      W_hh.T, b_hh.reshape(1, -1),
                      W1, b1.reshape(1, -1), W2, b2.reshape(1, -1),
                      e, t_steps, hg, hdec, be)
  return out_et.T


# speedup vs baseline: 1.4737x; 1.4737x over previous
import functools

import jax
import jax.numpy as jnp
from jax import lax
from jax.experimental import pallas as pl
from jax.experimental.pallas import tpu as pltpu
from jax.experimental.pallas import tpu_sc as plsc

_NC, _NS = 2, 16
_NW = _NC * _NS
_CB = 128


def _mesh():
  return plsc.VectorSubcoreMesh(core_axis_name="c", subcore_axis_name="s",
                                num_cores=_NC, num_subcores=_NS)


_NB = 2


@functools.cache
def _segsum(n_rows: int, feat: int, e_pad: int, tc_tiling: bool = True):
  cb = _CB
  rpw = -(-(n_rows + 1) // (_NS * 8)) * 8
  npad = rpw * _NS
  cpw = e_pad // (_NW * cb)
  assert cpw % _NB == 0 and cpw > _NB
  sz_last = n_rows - (_NS - 1) * rpw

  @functools.partial(
      pl.kernel, mesh=_mesh(),
      out_type=jax.ShapeDtypeStruct((_NC, n_rows, feat), jnp.float32),
      compiler_params=pltpu.CompilerParams(use_tc_tiling_on_sc=tc_tiling),
      scratch_types=[
          pltpu.VMEM((cpw, cb), jnp.int32),
          pltpu.VMEM((cpw, cb), jnp.int32),
          [pltpu.VMEM((cb, feat), jnp.float32)] * _NB,
          pltpu.VMEM_SHARED((npad, feat), jnp.float32),
          [pltpu.SemaphoreType.DMA] * _NB,
      ],
  )
  def seg(u_hbm, gidx_hbm, sidx_hbm, zeros_hbm, out_hbm,
          gv, sv, rows_v, acc, gsem):
    c = lax.axis_index("c")
    s = lax.axis_index("s")
    w = c * _NS + s
    pltpu.sync_copy(zeros_hbm, acc.at[pl.ds(s * rpw, rpw)])
    cr0 = pl.multiple_of(w * cpw, 8)
    pltpu.sync_copy(gidx_hbm.at[pl.ds(cr0, cpw)], gv)
    pltpu.sync_copy(sidx_hbm.at[pl.ds(cr0, cpw)], sv)
    plsc.subcore_barrier()

    for b in range(_NB):
      pltpu.async_copy(u_hbm.at[gv.at[b]], rows_v[b], gsem[b])

    def block(blk, carry):
      j0 = blk * _NB
      for b in range(_NB):
        j = j0 + b
        pltpu.make_async_copy(u_hbm.at[gv.at[b]], rows_v[b], gsem[b]).wait()
        pltpu.sync_copy(rows_v[b], acc.at[sv.at[j]], add=True)

        @pl.when(j + _NB < cpw)
        def _():
          pltpu.async_copy(u_hbm.at[gv.at[j + _NB]], rows_v[b], gsem[b])

      return carry

    lax.fori_loop(0, cpw // _NB, block, 0)
    plsc.subcore_barrier()

    @pl.when(s < _NS - 1)
    def _():
      r0 = s * rpw
      pltpu.sync_copy(acc.at[pl.ds(r0, rpw)], out_hbm.at[c].at[pl.ds(r0, rpw)])

    @pl.when(s == _NS - 1)
    def _():
      r0 = (_NS - 1) * rpw
      pltpu.sync_copy(acc.at[pl.ds(r0, sz_last)],
                      out_hbm.at[c].at[pl.ds(r0, sz_last)])

  return seg


@functools.cache
def _degree(n_rows: int, feat: int, e_pad: int):
  cb = _CB
  rpw = -(-(n_rows + 1) // (_NS * 8)) * 8
  npad = rpw * _NS
  cpw = e_pad // (_NW * cb)
  sz_last = n_rows - (_NS - 1) * rpw

  @functools.partial(
      pl.kernel, mesh=_mesh(),
      out_type=jax.ShapeDtypeStruct((_NC, n_rows, feat), jnp.float32),
      compiler_params=pltpu.CompilerParams(use_tc_tiling_on_sc=False),
      scratch_types=[
          pltpu.VMEM((cpw, cb), jnp.int32),
          pltpu.VMEM((cb, feat), jnp.float32),
          pltpu.VMEM_SHARED((npad, feat), jnp.float32),
      ],
  )
  def deg(sidx_hbm, ones_hbm, zeros_hbm, out_hbm, sv, ones_v, acc):
    c = lax.axis_index("c")
    s = lax.axis_index("s")
    w = c * _NS + s
    pltpu.sync_copy(ones_hbm, ones_v)
    pltpu.sync_copy(zeros_hbm, acc.at[pl.ds(s * rpw, rpw)])
    cr0 = pl.multiple_of(w * cpw, 8)
    pltpu.sync_copy(sidx_hbm.at[pl.ds(cr0, cpw)], sv)
    plsc.subcore_barrier()

    def chunk(j, carry):
      pltpu.sync_copy(ones_v, acc.at[sv.at[j]], add=True)
      return carry

    lax.fori_loop(0, cpw, chunk, 0)
    plsc.subcore_barrier()

    @pl.when(s < _NS - 1)
    def _():
      r0 = s * rpw
      pltpu.sync_copy(acc.at[pl.ds(r0, rpw)], out_hbm.at[c].at[pl.ds(r0, rpw)])

    @pl.when(s == _NS - 1)
    def _():
      r0 = (_NS - 1) * rpw
      pltpu.sync_copy(acc.at[pl.ds(r0, sz_last)],
                      out_hbm.at[c].at[pl.ds(r0, sz_last)])

  return deg


@functools.cache
def _gather2(n_rows: int, feat: int, e_pad: int, tc_tiling: bool = True):
  cb = _CB
  cpw = e_pad // (_NW * cb)

  @functools.partial(
      pl.kernel, mesh=_mesh(),
      out_type=(jax.ShapeDtypeStruct((e_pad, feat), jnp.float32),
                jax.ShapeDtypeStruct((e_pad, feat), jnp.float32)),
      compiler_params=pltpu.CompilerParams(use_tc_tiling_on_sc=tc_tiling),
      scratch_types=[
          pltpu.VMEM((cpw, cb), jnp.int32),
          pltpu.VMEM((cpw, cb), jnp.int32),
          [pltpu.VMEM((cb, feat), jnp.float32)] * 2,
          [pltpu.VMEM((cb, feat), jnp.float32)] * 2,
          [pltpu.SemaphoreType.DMA] * 2,
          [pltpu.SemaphoreType.DMA] * 2,
      ],
  )
  def gat(tab_hbm, aidx_hbm, bidx_hbm, oa_hbm, ob_hbm,
          av, bv, rows_a, rows_b, gsa, gsb):
    c = lax.axis_index("c")
    s = lax.axis_index("s")
    w = c * _NS + s
    w_base = pl.multiple_of(w * (cpw * cb), 8)
    cr0 = pl.multiple_of(w * cpw, 8)
    pltpu.sync_copy(aidx_hbm.at[pl.ds(cr0, cpw)], av)
    pltpu.sync_copy(bidx_hbm.at[pl.ds(cr0, cpw)], bv)
    for b in range(2):
      pltpu.async_copy(tab_hbm.at[av.at[b]], rows_a[b], gsa[b])
      pltpu.async_copy(tab_hbm.at[bv.at[b]], rows_b[b], gsb[b])

    def block(blk, carry):
      j0 = blk * 2
      for b in range(2):
        j = j0 + b
        base = pl.multiple_of(w_base + j * cb, 8)
        pltpu.make_async_copy(tab_hbm.at[av.at[b]], rows_a[b], gsa[b]).wait()
        pltpu.sync_copy(rows_a[b], oa_hbm.at[pl.ds(base, cb)])
        pltpu.make_async_copy(tab_hbm.at[bv.at[b]], rows_b[b], gsb[b]).wait()
        pltpu.sync_copy(rows_b[b], ob_hbm.at[pl.ds(base, cb)])

        @pl.when(j + 2 < cpw)
        def _():
          pltpu.async_copy(tab_hbm.at[av.at[j + 2]], rows_a[b], gsa[b])
          pltpu.async_copy(tab_hbm.at[bv.at[j + 2]], rows_b[b], gsb[b])

      return carry

    lax.fori_loop(0, cpw // 2, block, 0)

  return gat


def _vspec(bn, *trail):
  return pl.BlockSpec((bn,) + trail, lambda i: (i,) + (0,) * len(trail))


def _wspec(shape):
  return pl.BlockSpec(shape, lambda i: (0,) * len(shape))


def _k_prescale(degp, x_seq, bn):
  t, n, f = x_seq.shape
  fd = degp.shape[2]

  def body(dp_ref, x_ref, dv_ref, xs_ref):
    deg = dp_ref[0, :, 0] + dp_ref[1, :, 0]
    dv = jnp.where(deg > 0, 1.0 / jnp.sqrt(jnp.maximum(deg, 1e-12)), 0.0)
    dv_ref[...] = dv[:, None]
    xs_ref[...] = x_ref[...] * dv[None, :, None]

  return pl.pallas_call(
      body,
      grid=(n // bn,),
      in_specs=[pl.BlockSpec((2, bn, fd), lambda i: (0, i, 0)),
                pl.BlockSpec((t, bn, f), lambda i: (0, i, 0))],
      out_specs=[_vspec(bn, 1), pl.BlockSpec((t, bn, f), lambda i: (0, i, 0))],
      out_shape=[jax.ShapeDtypeStruct((n, 1), jnp.float32),
                 jax.ShapeDtypeStruct((t, n, f), jnp.float32)],
  )(degp, x_seq)


def _k_mid(p1, dinv, bn):
  _, n, f = p1.shape

  def body(p_ref, dv_ref, lap_ref, m1_ref):
    ps = p_ref[0] + p_ref[1]
    dv = dv_ref[...]
    lap_ref[...] = -dv * ps
    m1_ref[...] = (dv * dv) * ps

  return pl.pallas_call(
      body,
      grid=(n // bn,),
      in_specs=[pl.BlockSpec((2, bn, f), lambda i: (0, i, 0)), _vspec(bn, 1)],
      out_specs=[_vspec(bn, f), _vspec(bn, f)],
      out_shape=[jax.ShapeDtypeStruct((n, f), jnp.float32)] * 2,
  )(p1, dinv)


def _k_cheb(v, lapv, p2, dinv, a0m2, a1, a2x2, bias, bn, t_idx):
  _, n, f = v.shape
  v_spec = pl.BlockSpec((1, bn, f), lambda i, t=t_idx: (t, i, 0))
  fo = a1.shape[1]

  def body(v_ref, lap_ref, p2_ref, dv_ref, a0_ref, a1_ref, a2_ref, b_ref,
           o_ref):
    vv = v_ref[...].reshape(bn, f)
    dv = dv_ref[...]
    l2 = dv * (p2_ref[0] + p2_ref[1])
    acc = jnp.dot(vv, a0_ref[...], preferred_element_type=jnp.float32)
    acc += jnp.dot(lap_ref[...], a1_ref[...], preferred_element_type=jnp.float32)
    acc += jnp.dot(l2, a2_ref[...], preferred_element_type=jnp.float32)
    o_ref[...] = acc + b_ref[...]

  return pl.pallas_call(
      body,
      grid=(n // bn,),
      in_specs=[v_spec, _vspec(bn, f),
                pl.BlockSpec((2, bn, f), lambda i: (0, i, 0)), _vspec(bn, 1),
                _wspec(a0m2.shape), _wspec(a1.shape), _wspec(a2x2.shape),
                _wspec(bias.shape)],
      out_specs=_vspec(bn, fo),
      out_shape=jax.ShapeDtypeStruct((n, fo), jnp.float32),
  )(v, lapv, p2, dinv, a0m2, a1, a2x2, bias)


def _k_t0(cx0, dinv, b_hz, b_hn, hg, bn):
  n = cx0.shape[0]

  def body(cx_ref, dv_ref, bz_ref, bnn_ref, h_ref, u_ref):
    cx = cx_ref[...]
    z = jax.nn.sigmoid(cx[:, :hg] + bz_ref[...])
    htil = jnp.tanh(cx[:, 2 * hg:] + bnn_ref[...])
    h1 = (1.0 - z) * htil
    h_ref[...] = h1
    u_ref[...] = dv_ref[...] * h1

  return pl.pallas_call(
      body,
      grid=(n // bn,),
      in_specs=[_vspec(bn, 3 * hg), _vspec(bn, 1), _wspec(b_hz.shape),
                _wspec(b_hn.shape)],
      out_specs=[_vspec(bn, hg), _vspec(bn, hg)],
      out_shape=[jax.ShapeDtypeStruct((n, hg), jnp.float32)] * 2,
  )(cx0, dinv, b_hz, b_hn)


def _k_zr(cx, h, laph, p2h, dinv, b0m2, b1w, b2x2, bzr, hg, bn):
  n = cx.shape[0]

  def body(cx_ref, h_ref, lap_ref, p2_ref, dv_ref, b0_ref, b1_ref, b2_ref,
           bb_ref, z_ref, hr_ref, u_ref):
    dv = dv_ref[...]
    hh = h_ref[...]
    l2 = dv * (p2_ref[0] + p2_ref[1])
    ch = jnp.dot(hh, b0_ref[...], preferred_element_type=jnp.float32)
    ch += jnp.dot(lap_ref[...], b1_ref[...],
                  preferred_element_type=jnp.float32)
    ch += jnp.dot(l2, b2_ref[...], preferred_element_type=jnp.float32)
    ch += bb_ref[...]
    cxv = cx_ref[...]
    z = jax.nn.sigmoid(cxv[:, :hg] + ch[:, :hg])
    r = jax.nn.sigmoid(cxv[:, hg:2 * hg] + ch[:, hg:])
    hr = hh * r
    z_ref[...] = z
    hr_ref[...] = hr
    u_ref[...] = dv * hr

  return pl.pallas_call(
      body,
      grid=(n // bn,),
      in_specs=[_vspec(bn, 3 * hg), _vspec(bn, hg), _vspec(bn, hg),
                pl.BlockSpec((2, bn, hg), lambda i: (0, i, 0)), _vspec(bn, 1),
                _wspec(b0m2.shape), _wspec(b1w.shape), _wspec(b2x2.shape),
                _wspec(bzr.shape)],
      out_specs=[_vspec(bn, hg), _vspec(bn, hg), _vspec(bn, hg)],
      out_shape=[jax.ShapeDtypeStruct((n, hg), jnp.float32)] * 3,
  )(cx, h, laph, p2h, dinv, b0m2, b1w, b2x2, bzr)


def _k_upd(cx, h, z, hr, lapn, p2n, dinv, c0m2, c1w, c2x2, bnn, hg, bn):
  n = cx.shape[0]

  def body(cx_ref, h_ref, z_ref, hr_ref, lap_ref, p2_ref, dv_ref, c0_ref,
           c1_ref, c2_ref, bb_ref, hn_ref, u_ref):
    dv = dv_ref[...]
    l2 = dv * (p2_ref[0] + p2_ref[1])
    ch = jnp.dot(hr_ref[...], c0_ref[...], preferred_element_type=jnp.float32)
    ch += jnp.dot(lap_ref[...], c1_ref[...],
                  preferred_element_type=jnp.float32)
    ch += jnp.dot(l2, c2_ref[...], preferred_element_type=jnp.float32)
    htil = jnp.tanh(cx_ref[...][:, 2 * hg:] + ch + bb_ref[...])
    z = z_ref[...]
    hn = z * h_ref[...] + (1.0 - z) * htil
    hn_ref[...] = hn
    u_ref[...] = dv * hn

  return pl.pallas_call(
      body,
      grid=(n // bn,),
      in_specs=[_vspec(bn, 3 * hg), _vspec(bn, hg), _vspec(bn, hg),
                _vspec(bn, hg), _vspec(bn, hg),
                pl.BlockSpec((2, bn, hg), lambda i: (0, i, 0)), _vspec(bn, 1),
                _wspec(c0m2.shape), _wspec(c1w.shape), _wspec(c2x2.shape),
                _wspec(bnn.shape)],
      out_specs=[_vspec(bn, hg), _vspec(bn, hg)],
      out_shape=[jax.ShapeDtypeStruct((n, hg), jnp.float32)] * 2,
  )(cx, h, z, hr, lapn, p2n, dinv, c0m2, c1w, c2x2, bnn)


def _k_decoder(hs, hd, tc16, st, w_s, w_d, w_tblk, w_st, b_ih, w_hh, b_hh,
               w1, b1, w2, b2, e, t_steps, hg, hdec, be):
  hp = hs.shape[1]
  fs = w_st.shape[0]
  ftt = tc16.shape[1]

  def body(hs_ref, hd_ref, tc_ref, st_ref, ws_ref, wd_ref, wt_ref, wst_ref,
           bih_ref, whh_ref, bhh_ref, w1_ref, b1_ref, w2_ref, b2_ref, o_ref):
    gi_base = jnp.dot(hs_ref[...], ws_ref[...],
                      preferred_element_type=jnp.float32)
    gi_base += jnp.dot(hd_ref[...], wd_ref[...],
                       preferred_element_type=jnp.float32)
    gi_base += jnp.dot(st_ref[...], wst_ref[...],
                       preferred_element_type=jnp.float32)
    gi_base += bih_ref[...]
    gt_all = jnp.dot(tc_ref[...], wt_ref[...],
                     preferred_element_type=jnp.float32)
    h = jnp.zeros((be, hdec), jnp.float32)
    cols = []
    for t in range(t_steps):
      gi = gi_base + gt_all[:, 3 * hdec * t:3 * hdec * (t + 1)]
      gh = jnp.dot(h, whh_ref[...], preferred_element_type=jnp.float32)
      gh += bhh_ref[...]
      r = jax.nn.sigmoid(gi[:, :hdec] + gh[:, :hdec])
      z = jax.nn.sigmoid(gi[:, hdec:2 * hdec] + gh[:, hdec:2 * hdec])
      nn = jnp.tanh(gi[:, 2 * hdec:] + r * gh[:, 2 * hdec:])
      h = (1.0 - z) * nn + z * h
      hid = jax.nn.relu(
          jnp.dot(h, w1_ref[...], preferred_element_type=jnp.float32)
          + b1_ref[...])
      cols.append(jnp.dot(hid, w2_ref[...], preferred_element_type=jnp.float32)
                  + b2_ref[...])
    o_ref[...] = jnp.concatenate(cols, axis=1)

  return pl.pallas_call(
      body,
      grid=(e // be,),
      in_specs=[_vspec(be, hp), _vspec(be, hp), _vspec(be, ftt),
                _vspec(be, fs),
                _wspec(w_s.shape), _wspec(w_d.shape), _wspec(w_tblk.shape),
                _wspec(w_st.shape), _wspec(b_ih.shape), _wspec(w_hh.shape),
                _wspec(b_hh.shape), _wspec(w1.shape), _wspec(b1.shape),
                _wspec(w2.shape), _wspec(b2.shape)],
      out_specs=_vspec(be, t_steps),
      out_shape=jax.ShapeDtypeStruct((e, t_steps), jnp.float32),
  )(hs, hd, tc16, st, w_s, w_d, w_tblk, w_st, b_ih, w_hh, b_hh, w1, b1, w2, b2)


def kernel(x_seq, edge_index, time_seq, static_feats,
           W_xz, b_xz, W_hz, b_hz, W_xr, b_xr, W_hr, b_hr,
           W_xh, b_xh, W_hn, b_hn,
           W_ih, W_hh, b_ih, b_hh, W1, b1, W2, b2):
  t_steps, n, f_node = x_seq.shape
  e = edge_index.shape[1]
  hg = W_hz.shape[2]
  hdec = W_hh.shape[0] // 3
  ft = time_seq.shape[2]

  cpw = -(-e // (_NW * _CB))
  e_pad = _NW * cpw * _CB
  pad = e_pad - e
  bn = 2000
  be = 2000

  src = edge_index[0]
  dst = edge_index[1]
  zpad_i = jnp.zeros((pad,), jnp.int32)
  npad_i = jnp.full((pad,), n, jnp.int32)
  src_g = jnp.concatenate([src, zpad_i]).reshape(-1, _CB)
  dst_g = jnp.concatenate([dst, zpad_i]).reshape(-1, _CB)
  src_s = jnp.concatenate([src, npad_i]).reshape(-1, _CB)
  dst_s = jnp.concatenate([dst, npad_i]).reshape(-1, _CB)

  rpw = -(-(n + 1) // (_NS * 8)) * 8
  z_fn = jnp.zeros((rpw, f_node), jnp.float32)
  z_hg = jnp.zeros((rpw, hg), jnp.float32)
  ones_row = jnp.ones((_CB, 16), jnp.float32)

  seg_fn = _segsum(n, f_node, e_pad)
  seg_hg = _segsum(n, hg, e_pad, tc_tiling=False)
  deg_k = _degree(n, 16, e_pad)
  gat_k = _gather2(n, hg, e_pad, tc_tiling=False)

  degp = deg_k(src_s, ones_row, jnp.zeros((rpw, 16), jnp.float32))
  dinv, xs = _k_prescale(degp, x_seq, bn)

  a_k = [jnp.concatenate([W_xz[k], W_xr[k], W_xh[k]], axis=1) for k in range(3)]
  a0m2, a1w, a2x2 = a_k[0] - a_k[2], a_k[1], 2.0 * a_k[2]
  bx = jnp.concatenate([b_xz, b_xr, b_xh]).reshape(1, 3 * hg)
  b_k = [jnp.concatenate([W_hz[k], W_hr[k]], axis=1) for k in range(3)]
  b0m2, b1w, b2x2 = b_k[0] - b_k[2], b_k[1], 2.0 * b_k[2]
  bzr = jnp.concatenate([b_hz, b_hr]).reshape(1, 2 * hg)
  c0m2, c1w, c2x2 = W_hn[0] - W_hn[2], W_hn[1], 2.0 * W_hn[2]
  bz2 = b_hz.reshape(1, hg)
  bn2 = b_hn.reshape(1, hg)

  cxs = []
  for t in range(t_steps):
    p1 = seg_fn(xs[t], src_g, dst_s, z_fn)
    lapx, m1x = _k_mid(p1, dinv, bn)
    p2 = seg_fn(m1x, src_g, dst_s, z_fn)
    cxs.append(_k_cheb(x_seq, lapx, p2, dinv, a0m2, a1w, a2x2, bx, bn, t))

  h_cur, u_cur = _k_t0(cxs[0], dinv, bz2, bn2, hg, bn)
  for t in range(1, t_steps):
    p1h = seg_hg(u_cur, src_g, dst_s, z_hg)
    laph, m1h = _k_mid(p1h, dinv, bn)
    p2h = seg_hg(m1h, src_g, dst_s, z_hg)
    z_gate, hr, uhr = _k_zr(cxs[t], h_cur, laph, p2h, dinv, b0m2, b1w, b2x2,
                            bzr, hg, bn)
    p1n = seg_hg(uhr, src_g, dst_s, z_hg)
    lapn, m1n = _k_mid(p1n, dinv, bn)
    p2n = seg_hg(m1n, src_g, dst_s, z_hg)
    h_cur, u_cur = _k_upd(cxs[t], h_cur, z_gate, hr, lapn, p2n, dinv, c0m2,
                          c1w, c2x2, bn2, hg, bn)

  hs, hd = gat_k(h_cur, src_g, dst_g)
  tc16 = jnp.transpose(time_seq, (1, 0, 2)).reshape(e, t_steps * ft)
  w_s = W_ih[:, :hg].T
  w_d = W_ih[:, hg:2 * hg].T
  w_t = W_ih[:, 2 * hg:2 * hg + ft].T
  w_st = W_ih[:, 2 * hg + ft:].T
  w_tblk = jnp.zeros((t_steps * ft, t_steps * 3 * hdec), jnp.float32)
  for t in range(t_steps):
    w_tblk = w_tblk.at[ft * t:ft * (t + 1),
                       3 * hdec * t:3 * hdec * (t + 1)].set(w_t)
  out_et = _k_decoder(hs, hd, tc16, static_feats,
                      w_s, w_d, w_tblk, w_st, b_ih.reshape(1, -1),
                      W_hh.T, b_hh.reshape(1, -1),
                      W1, b1.reshape(1, -1), W2, b2.reshape(1, -1),
                      e, t_steps, hg, hdec, be)
  return out_et.T


# speedup vs baseline: 1.8507x; 1.2559x over previous
import functools

import jax
import jax.numpy as jnp
from jax import lax
from jax.experimental import pallas as pl
from jax.experimental.pallas import tpu as pltpu
from jax.experimental.pallas import tpu_sc as plsc

_NC, _NS = 2, 16
_NW = _NC * _NS
_CB = 128


def _mesh():
  return plsc.VectorSubcoreMesh(core_axis_name="c", subcore_axis_name="s",
                                num_cores=_NC, num_subcores=_NS)


_NB = 2


@functools.cache
def _segsum(n_rows: int, feat: int, e_pad: int, tc_tiling: bool = True,
            dtype=jnp.float32, nb: int = _NB):
  cb = _CB
  rpw = -(-(n_rows + 1) // (_NS * 8)) * 8
  npad = rpw * _NS
  cpw = e_pad // (_NW * cb)
  assert cpw % nb == 0 and cpw > nb
  sz_last = n_rows - (_NS - 1) * rpw

  @functools.partial(
      pl.kernel, mesh=_mesh(),
      out_type=jax.ShapeDtypeStruct((_NC, n_rows, feat), dtype),
      compiler_params=pltpu.CompilerParams(use_tc_tiling_on_sc=tc_tiling),
      scratch_types=[
          pltpu.VMEM((cpw, cb), jnp.int32),
          pltpu.VMEM((cpw, cb), jnp.int32),
          [pltpu.VMEM((cb, feat), dtype)] * nb,
          pltpu.VMEM_SHARED((npad, feat), dtype),
          [pltpu.SemaphoreType.DMA] * nb,
      ],
  )
  def seg(u_hbm, gidx_hbm, sidx_hbm, zeros_hbm, out_hbm,
          gv, sv, rows_v, acc, gsem):
    c = lax.axis_index("c")
    s = lax.axis_index("s")
    w = c * _NS + s
    pltpu.sync_copy(zeros_hbm, acc.at[pl.ds(s * rpw, rpw)])
    cr0 = pl.multiple_of(w * cpw, 8)
    pltpu.sync_copy(gidx_hbm.at[pl.ds(cr0, cpw)], gv)
    pltpu.sync_copy(sidx_hbm.at[pl.ds(cr0, cpw)], sv)
    plsc.subcore_barrier()

    for b in range(nb):
      pltpu.async_copy(u_hbm.at[gv.at[b]], rows_v[b], gsem[b])

    def block(blk, carry):
      j0 = blk * nb
      for b in range(nb):
        j = j0 + b
        pltpu.make_async_copy(u_hbm.at[gv.at[b]], rows_v[b], gsem[b]).wait()
        pltpu.sync_copy(rows_v[b], acc.at[sv.at[j]], add=True)

        @pl.when(j + nb < cpw)
        def _():
          pltpu.async_copy(u_hbm.at[gv.at[j + nb]], rows_v[b], gsem[b])

      return carry

    lax.fori_loop(0, cpw // nb, block, 0)
    plsc.subcore_barrier()

    @pl.when(s < _NS - 1)
    def _():
      r0 = s * rpw
      pltpu.sync_copy(acc.at[pl.ds(r0, rpw)], out_hbm.at[c].at[pl.ds(r0, rpw)])

    @pl.when(s == _NS - 1)
    def _():
      r0 = (_NS - 1) * rpw
      pltpu.sync_copy(acc.at[pl.ds(r0, sz_last)],
                      out_hbm.at[c].at[pl.ds(r0, sz_last)])

  return seg


@functools.cache
def _degree(n_rows: int, feat: int, e_pad: int):
  cb = _CB
  rpw = -(-(n_rows + 1) // (_NS * 8)) * 8
  npad = rpw * _NS
  cpw = e_pad // (_NW * cb)
  sz_last = n_rows - (_NS - 1) * rpw

  @functools.partial(
      pl.kernel, mesh=_mesh(),
      out_type=jax.ShapeDtypeStruct((_NC, n_rows, feat), jnp.float32),
      compiler_params=pltpu.CompilerParams(use_tc_tiling_on_sc=False),
      scratch_types=[
          pltpu.VMEM((cpw, cb), jnp.int32),
          pltpu.VMEM((cb, feat), jnp.float32),
          pltpu.VMEM_SHARED((npad, feat), jnp.float32),
      ],
  )
  def deg(sidx_hbm, ones_hbm, zeros_hbm, out_hbm, sv, ones_v, acc):
    c = lax.axis_index("c")
    s = lax.axis_index("s")
    w = c * _NS + s
    pltpu.sync_copy(ones_hbm, ones_v)
    pltpu.sync_copy(zeros_hbm, acc.at[pl.ds(s * rpw, rpw)])
    cr0 = pl.multiple_of(w * cpw, 8)
    pltpu.sync_copy(sidx_hbm.at[pl.ds(cr0, cpw)], sv)
    plsc.subcore_barrier()

    def chunk(j, carry):
      pltpu.sync_copy(ones_v, acc.at[sv.at[j]], add=True)
      return carry

    lax.fori_loop(0, cpw, chunk, 0)
    plsc.subcore_barrier()

    @pl.when(s < _NS - 1)
    def _():
      r0 = s * rpw
      pltpu.sync_copy(acc.at[pl.ds(r0, rpw)], out_hbm.at[c].at[pl.ds(r0, rpw)])

    @pl.when(s == _NS - 1)
    def _():
      r0 = (_NS - 1) * rpw
      pltpu.sync_copy(acc.at[pl.ds(r0, sz_last)],
                      out_hbm.at[c].at[pl.ds(r0, sz_last)])

  return deg


@functools.cache
def _gather2(n_rows: int, feat: int, e_pad: int, tc_tiling: bool = True):
  cb = _CB
  cpw = e_pad // (_NW * cb)

  @functools.partial(
      pl.kernel, mesh=_mesh(),
      out_type=(jax.ShapeDtypeStruct((e_pad, feat), jnp.float32),
                jax.ShapeDtypeStruct((e_pad, feat), jnp.float32)),
      compiler_params=pltpu.CompilerParams(use_tc_tiling_on_sc=tc_tiling),
      scratch_types=[
          pltpu.VMEM((cpw, cb), jnp.int32),
          pltpu.VMEM((cpw, cb), jnp.int32),
          [pltpu.VMEM((cb, feat), jnp.float32)] * 2,
          [pltpu.VMEM((cb, feat), jnp.float32)] * 2,
          [pltpu.SemaphoreType.DMA] * 2,
          [pltpu.SemaphoreType.DMA] * 2,
      ],
  )
  def gat(tab_hbm, aidx_hbm, bidx_hbm, oa_hbm, ob_hbm,
          av, bv, rows_a, rows_b, gsa, gsb):
    c = lax.axis_index("c")
    s = lax.axis_index("s")
    w = c * _NS + s
    w_base = pl.multiple_of(w * (cpw * cb), 8)
    cr0 = pl.multiple_of(w * cpw, 8)
    pltpu.sync_copy(aidx_hbm.at[pl.ds(cr0, cpw)], av)
    pltpu.sync_copy(bidx_hbm.at[pl.ds(cr0, cpw)], bv)
    for b in range(2):
      pltpu.async_copy(tab_hbm.at[av.at[b]], rows_a[b], gsa[b])
      pltpu.async_copy(tab_hbm.at[bv.at[b]], rows_b[b], gsb[b])

    def block(blk, carry):
      j0 = blk * 2
      for b in range(2):
        j = j0 + b
        base = pl.multiple_of(w_base + j * cb, 8)
        pltpu.make_async_copy(tab_hbm.at[av.at[b]], rows_a[b], gsa[b]).wait()
        pltpu.sync_copy(rows_a[b], oa_hbm.at[pl.ds(base, cb)])
        pltpu.make_async_copy(tab_hbm.at[bv.at[b]], rows_b[b], gsb[b]).wait()
        pltpu.sync_copy(rows_b[b], ob_hbm.at[pl.ds(base, cb)])

        @pl.when(j + 2 < cpw)
        def _():
          pltpu.async_copy(tab_hbm.at[av.at[j + 2]], rows_a[b], gsa[b])
          pltpu.async_copy(tab_hbm.at[bv.at[j + 2]], rows_b[b], gsb[b])

      return carry

    lax.fori_loop(0, cpw // 2, block, 0)

  return gat


def _vspec(bn, *trail):
  return pl.BlockSpec((bn,) + trail, lambda i: (i,) + (0,) * len(trail))


def _wspec(shape):
  return pl.BlockSpec(shape, lambda i: (0,) * len(shape))


def _k_prescale(degp, x_seq, bn):
  t, n, f = x_seq.shape
  fd = degp.shape[2]

  def body(dp_ref, x_ref, dv_ref, xs_ref):
    deg = dp_ref[0, :, 0] + dp_ref[1, :, 0]
    dv = jnp.where(deg > 0, 1.0 / jnp.sqrt(jnp.maximum(deg, 1e-12)), 0.0)
    dv_ref[...] = dv[:, None]
    xs_ref[...] = (x_ref[...] * dv[None, :, None]).astype(xs_ref.dtype)

  return pl.pallas_call(
      body,
      grid=(n // bn,),
      in_specs=[pl.BlockSpec((2, bn, fd), lambda i: (0, i, 0)),
                pl.BlockSpec((t, bn, f), lambda i: (0, i, 0))],
      out_specs=[_vspec(bn, 1), pl.BlockSpec((t, bn, f), lambda i: (0, i, 0))],
      out_shape=[jax.ShapeDtypeStruct((n, 1), jnp.float32),
                 jax.ShapeDtypeStruct((t, n, f), jnp.bfloat16)],
  )(degp, x_seq)


def _k_mid(p1, dinv, bn, m1_dtype=jnp.float32):
  _, n, f = p1.shape

  def body(p_ref, dv_ref, lap_ref, m1_ref):
    ps = p_ref[0].astype(jnp.float32) + p_ref[1].astype(jnp.float32)
    dv = dv_ref[...]
    lap_ref[...] = -dv * ps
    m1_ref[...] = ((dv * dv) * ps).astype(m1_ref.dtype)

  return pl.pallas_call(
      body,
      grid=(n // bn,),
      in_specs=[pl.BlockSpec((2, bn, f), lambda i: (0, i, 0)), _vspec(bn, 1)],
      out_specs=[_vspec(bn, f), _vspec(bn, f)],
      out_shape=[jax.ShapeDtypeStruct((n, f), jnp.float32),
                 jax.ShapeDtypeStruct((n, f), m1_dtype)],
  )(p1, dinv)


def _k_cheb(v, lapv, p2, dinv, a0m2, a1, a2x2, bias, bn, t_idx):
  _, n, f = v.shape
  v_spec = pl.BlockSpec((1, bn, f), lambda i, t=t_idx: (t, i, 0))
  fo = a1.shape[1]

  def body(v_ref, lap_ref, p2_ref, dv_ref, a0_ref, a1_ref, a2_ref, b_ref,
           o_ref):
    vv = v_ref[...].reshape(bn, f)
    dv = dv_ref[...]
    l2 = dv * (p2_ref[0].astype(jnp.float32) + p2_ref[1].astype(jnp.float32))
    acc = jnp.dot(vv, a0_ref[...], preferred_element_type=jnp.float32)
    acc += jnp.dot(lap_ref[...], a1_ref[...], preferred_element_type=jnp.float32)
    acc += jnp.dot(l2, a2_ref[...], preferred_element_type=jnp.float32)
    o_ref[...] = acc + b_ref[...]

  return pl.pallas_call(
      body,
      grid=(n // bn,),
      in_specs=[v_spec, _vspec(bn, f),
                pl.BlockSpec((2, bn, f), lambda i: (0, i, 0)), _vspec(bn, 1),
                _wspec(a0m2.shape), _wspec(a1.shape), _wspec(a2x2.shape),
                _wspec(bias.shape)],
      out_specs=_vspec(bn, fo),
      out_shape=jax.ShapeDtypeStruct((n, fo), jnp.float32),
  )(v, lapv, p2, dinv, a0m2, a1, a2x2, bias)


def _k_t0(cx0, dinv, b_hz, b_hn, hg, bn):
  n = cx0.shape[0]

  def body(cx_ref, dv_ref, bz_ref, bnn_ref, h_ref, u_ref):
    cx = cx_ref[...]
    z = jax.nn.sigmoid(cx[:, :hg] + bz_ref[...])
    htil = jnp.tanh(cx[:, 2 * hg:] + bnn_ref[...])
    h1 = (1.0 - z) * htil
    h_ref[...] = h1
    u_ref[...] = dv_ref[...] * h1

  return pl.pallas_call(
      body,
      grid=(n // bn,),
      in_specs=[_vspec(bn, 3 * hg), _vspec(bn, 1), _wspec(b_hz.shape),
                _wspec(b_hn.shape)],
      out_specs=[_vspec(bn, hg), _vspec(bn, hg)],
      out_shape=[jax.ShapeDtypeStruct((n, hg), jnp.float32)] * 2,
  )(cx0, dinv, b_hz, b_hn)


def _k_zr(cx, h, laph, p2h, dinv, b0m2, b1w, b2x2, bzr, hg, bn):
  n = cx.shape[0]

  def body(cx_ref, h_ref, lap_ref, p2_ref, dv_ref, b0_ref, b1_ref, b2_ref,
           bb_ref, z_ref, hr_ref, u_ref):
    dv = dv_ref[...]
    hh = h_ref[...]
    l2 = dv * (p2_ref[0] + p2_ref[1])
    ch = jnp.dot(hh, b0_ref[...], preferred_element_type=jnp.float32)
    ch += jnp.dot(lap_ref[...], b1_ref[...],
                  preferred_element_type=jnp.float32)
    ch += jnp.dot(l2, b2_ref[...], preferred_element_type=jnp.float32)
    ch += bb_ref[...]
    cxv = cx_ref[...]
    z = jax.nn.sigmoid(cxv[:, :hg] + ch[:, :hg])
    r = jax.nn.sigmoid(cxv[:, hg:2 * hg] + ch[:, hg:])
    hr = hh * r
    z_ref[...] = z
    hr_ref[...] = hr
    u_ref[...] = dv * hr

  return pl.pallas_call(
      body,
      grid=(n // bn,),
      in_specs=[_vspec(bn, 3 * hg), _vspec(bn, hg), _vspec(bn, hg),
                pl.BlockSpec((2, bn, hg), lambda i: (0, i, 0)), _vspec(bn, 1),
                _wspec(b0m2.shape), _wspec(b1w.shape), _wspec(b2x2.shape),
                _wspec(bzr.shape)],
      out_specs=[_vspec(bn, hg), _vspec(bn, hg), _vspec(bn, hg)],
      out_shape=[jax.ShapeDtypeStruct((n, hg), jnp.float32)] * 3,
  )(cx, h, laph, p2h, dinv, b0m2, b1w, b2x2, bzr)


def _k_upd(cx, h, z, hr, lapn, p2n, dinv, c0m2, c1w, c2x2, bnn, hg, bn):
  n = cx.shape[0]

  def body(cx_ref, h_ref, z_ref, hr_ref, lap_ref, p2_ref, dv_ref, c0_ref,
           c1_ref, c2_ref, bb_ref, hn_ref, u_ref):
    dv = dv_ref[...]
    l2 = dv * (p2_ref[0] + p2_ref[1])
    ch = jnp.dot(hr_ref[...], c0_ref[...], preferred_element_type=jnp.float32)
    ch += jnp.dot(lap_ref[...], c1_ref[...],
                  preferred_element_type=jnp.float32)
    ch += jnp.dot(l2, c2_ref[...], preferred_element_type=jnp.float32)
    htil = jnp.tanh(cx_ref[...][:, 2 * hg:] + ch + bb_ref[...])
    z = z_ref[...]
    hn = z * h_ref[...] + (1.0 - z) * htil
    hn_ref[...] = hn
    u_ref[...] = dv * hn

  return pl.pallas_call(
      body,
      grid=(n // bn,),
      in_specs=[_vspec(bn, 3 * hg), _vspec(bn, hg), _vspec(bn, hg),
                _vspec(bn, hg), _vspec(bn, hg),
                pl.BlockSpec((2, bn, hg), lambda i: (0, i, 0)), _vspec(bn, 1),
                _wspec(c0m2.shape), _wspec(c1w.shape), _wspec(c2x2.shape),
                _wspec(bnn.shape)],
      out_specs=[_vspec(bn, hg), _vspec(bn, hg)],
      out_shape=[jax.ShapeDtypeStruct((n, hg), jnp.float32)] * 2,
  )(cx, h, z, hr, lapn, p2n, dinv, c0m2, c1w, c2x2, bnn)


def _k_decoder(hs, hd, tc16, st, w_s, w_d, w_tblk, w_st, b_ih, w_hh, b_hh,
               w1, b1, w2, b2, e, t_steps, hg, hdec, be):
  hp = hs.shape[1]
  fs = w_st.shape[0]
  ftt = tc16.shape[1]

  def body(hs_ref, hd_ref, tc_ref, st_ref, ws_ref, wd_ref, wt_ref, wst_ref,
           bih_ref, whh_ref, bhh_ref, w1_ref, b1_ref, w2_ref, b2_ref, o_ref):
    gi_base = jnp.dot(hs_ref[...], ws_ref[...],
                      preferred_element_type=jnp.float32)
    gi_base += jnp.dot(hd_ref[...], wd_ref[...],
                       preferred_element_type=jnp.float32)
    gi_base += jnp.dot(st_ref[...], wst_ref[...],
                       preferred_element_type=jnp.float32)
    gi_base += bih_ref[...]
    gt_all = jnp.dot(tc_ref[...], wt_ref[...],
                     preferred_element_type=jnp.float32)
    h = jnp.zeros((be, hdec), jnp.float32)
    cols = []
    for t in range(t_steps):
      gi = gi_base + gt_all[:, 3 * hdec * t:3 * hdec * (t + 1)]
      gh = jnp.dot(h, whh_ref[...], preferred_element_type=jnp.float32)
      gh += bhh_ref[...]
      r = jax.nn.sigmoid(gi[:, :hdec] + gh[:, :hdec])
      z = jax.nn.sigmoid(gi[:, hdec:2 * hdec] + gh[:, hdec:2 * hdec])
      nn = jnp.tanh(gi[:, 2 * hdec:] + r * gh[:, 2 * hdec:])
      h = (1.0 - z) * nn + z * h
      hid = jax.nn.relu(
          jnp.dot(h, w1_ref[...], preferred_element_type=jnp.float32)
          + b1_ref[...])
      cols.append(jnp.dot(hid, w2_ref[...], preferred_element_type=jnp.float32)
                  + b2_ref[...])
    o_ref[...] = jnp.concatenate(cols, axis=1)

  return pl.pallas_call(
      body,
      grid=(e // be,),
      in_specs=[_vspec(be, hp), _vspec(be, hp), _vspec(be, ftt),
                _vspec(be, fs),
                _wspec(w_s.shape), _wspec(w_d.shape), _wspec(w_tblk.shape),
                _wspec(w_st.shape), _wspec(b_ih.shape), _wspec(w_hh.shape),
                _wspec(b_hh.shape), _wspec(w1.shape), _wspec(b1.shape),
                _wspec(w2.shape), _wspec(b2.shape)],
      out_specs=_vspec(be, t_steps),
      out_shape=jax.ShapeDtypeStruct((e, t_steps), jnp.float32),
  )(hs, hd, tc16, st, w_s, w_d, w_tblk, w_st, b_ih, w_hh, b_hh, w1, b1, w2, b2)


def kernel(x_seq, edge_index, time_seq, static_feats,
           W_xz, b_xz, W_hz, b_hz, W_xr, b_xr, W_hr, b_hr,
           W_xh, b_xh, W_hn, b_hn,
           W_ih, W_hh, b_ih, b_hh, W1, b1, W2, b2):
  t_steps, n, f_node = x_seq.shape
  e = edge_index.shape[1]
  hg = W_hz.shape[2]
  hdec = W_hh.shape[0] // 3
  ft = time_seq.shape[2]

  cpw = -(-e // (_NW * _CB))
  e_pad = _NW * cpw * _CB
  pad = e_pad - e
  bn = 2000
  be = 2000

  src = edge_index[0]
  dst = edge_index[1]
  zpad_i = jnp.zeros((pad,), jnp.int32)
  npad_i = jnp.full((pad,), n, jnp.int32)
  src_g = jnp.concatenate([src, zpad_i]).reshape(-1, _CB)
  dst_g = jnp.concatenate([dst, zpad_i]).reshape(-1, _CB)
  src_s = jnp.concatenate([src, npad_i]).reshape(-1, _CB)
  dst_s = jnp.concatenate([dst, npad_i]).reshape(-1, _CB)

  rpw = -(-(n + 1) // (_NS * 8)) * 8
  z_fn = jnp.zeros((rpw, f_node), jnp.bfloat16)
  z_hg = jnp.zeros((rpw, hg), jnp.float32)
  ones_row = jnp.ones((_CB, 16), jnp.float32)

  seg_fn = _segsum(n, f_node, e_pad, tc_tiling=False, dtype=jnp.bfloat16,
                   nb=4)
  seg_hg = _segsum(n, hg, e_pad, tc_tiling=False, nb=4)
  deg_k = _degree(n, 16, e_pad)
  gat_k = _gather2(n, hg, e_pad, tc_tiling=False)

  degp = deg_k(src_s, ones_row, jnp.zeros((rpw, 16), jnp.float32))
  dinv, xs = _k_prescale(degp, x_seq, bn)

  a_k = [jnp.concatenate([W_xz[k], W_xr[k], W_xh[k]], axis=1) for k in range(3)]
  a0m2, a1w, a2x2 = a_k[0] - a_k[2], a_k[1], 2.0 * a_k[2]
  bx = jnp.concatenate([b_xz, b_xr, b_xh]).reshape(1, 3 * hg)
  b_k = [jnp.concatenate([W_hz[k], W_hr[k]], axis=1) for k in range(3)]
  b0m2, b1w, b2x2 = b_k[0] - b_k[2], b_k[1], 2.0 * b_k[2]
  bzr = jnp.concatenate([b_hz, b_hr]).reshape(1, 2 * hg)
  c0m2, c1w, c2x2 = W_hn[0] - W_hn[2], W_hn[1], 2.0 * W_hn[2]
  bz2 = b_hz.reshape(1, hg)
  bn2 = b_hn.reshape(1, hg)

  cxs = []
  for t in range(t_steps):
    p1 = seg_fn(xs[t], src_g, dst_s, z_fn)
    lapx, m1x = _k_mid(p1, dinv, bn, m1_dtype=jnp.bfloat16)
    p2 = seg_fn(m1x, src_g, dst_s, z_fn)
    cxs.append(_k_cheb(x_seq, lapx, p2, dinv, a0m2, a1w, a2x2, bx, bn, t))

  h_cur, u_cur = _k_t0(cxs[0], dinv, bz2, bn2, hg, bn)
  for t in range(1, t_steps):
    p1h = seg_hg(u_cur, src_g, dst_s, z_hg)
    laph, m1h = _k_mid(p1h, dinv, bn)
    p2h = seg_hg(m1h, src_g, dst_s, z_hg)
    z_gate, hr, uhr = _k_zr(cxs[t], h_cur, laph, p2h, dinv, b0m2, b1w, b2x2,
                            bzr, hg, bn)
    p1n = seg_hg(uhr, src_g, dst_s, z_hg)
    lapn, m1n = _k_mid(p1n, dinv, bn)
    p2n = seg_hg(m1n, src_g, dst_s, z_hg)
    h_cur, u_cur = _k_upd(cxs[t], h_cur, z_gate, hr, lapn, p2n, dinv, c0m2,
                          c1w, c2x2, bn2, hg, bn)

  hs, hd = gat_k(h_cur, src_g, dst_g)
  tc16 = jnp.transpose(time_seq, (1, 0, 2)).reshape(e, t_steps * ft)
  w_s = W_ih[:, :hg].T
  w_d = W_ih[:, hg:2 * hg].T
  w_t = W_ih[:, 2 * hg:2 * hg + ft].T
  w_st = W_ih[:, 2 * hg + ft:].T
  w_tblk = jnp.zeros((t_steps * ft, t_steps * 3 * hdec), jnp.float32)
  for t in range(t_steps):
    w_tblk = w_tblk.at[ft * t:ft * (t + 1),
                       3 * hdec * t:3 * hdec * (t + 1)].set(w_t)
  out_et = _k_decoder(hs, hd, tc16, static_feats,
                      w_s, w_d, w_tblk, w_st, b_ih.reshape(1, -1),
                      W_hh.T, b_hh.reshape(1, -1),
                      W1, b1.reshape(1, -1), W2, b2.reshape(1, -1),
                      e, t_steps, hg, hdec, be)
  return out_et.T


# speedup vs baseline: 2.2890x; 1.2368x over previous
import functools

import jax
import jax.numpy as jnp
from jax import lax
from jax.experimental import pallas as pl
from jax.experimental.pallas import tpu as pltpu
from jax.experimental.pallas import tpu_sc as plsc

_NC, _NS = 2, 16
_NW = _NC * _NS
_CB = 128


def _mesh():
  return plsc.VectorSubcoreMesh(core_axis_name="c", subcore_axis_name="s",
                                num_cores=_NC, num_subcores=_NS)


_NB = 2


@functools.cache
def _segsum(n_rows: int, feat: int, e_pad: int, tc_tiling: bool = True,
            dtype=jnp.float32, nb: int = _NB):
  cb = _CB
  rpw = -(-(n_rows + 1) // (_NS * 8)) * 8
  npad = rpw * _NS
  cpw = e_pad // (_NW * cb)
  assert cpw % nb == 0 and cpw > nb
  sz_last = n_rows - (_NS - 1) * rpw

  @functools.partial(
      pl.kernel, mesh=_mesh(),
      out_type=jax.ShapeDtypeStruct((_NC, n_rows, feat), dtype),
      compiler_params=pltpu.CompilerParams(use_tc_tiling_on_sc=tc_tiling),
      scratch_types=[
          pltpu.VMEM((cpw, cb), jnp.int32),
          pltpu.VMEM((cpw, cb), jnp.int32),
          [pltpu.VMEM((cb, feat), dtype)] * nb,
          pltpu.VMEM_SHARED((npad, feat), dtype),
          [pltpu.SemaphoreType.DMA] * nb,
      ],
  )
  def seg(u_hbm, gidx_hbm, sidx_hbm, zeros_hbm, out_hbm,
          gv, sv, rows_v, acc, gsem):
    c = lax.axis_index("c")
    s = lax.axis_index("s")
    w = c * _NS + s
    pltpu.sync_copy(zeros_hbm, acc.at[pl.ds(s * rpw, rpw)])
    cr0 = pl.multiple_of(w * cpw, 8)
    pltpu.sync_copy(gidx_hbm.at[pl.ds(cr0, cpw)], gv)
    pltpu.sync_copy(sidx_hbm.at[pl.ds(cr0, cpw)], sv)
    plsc.subcore_barrier()

    for b in range(nb):
      pltpu.async_copy(u_hbm.at[gv.at[b]], rows_v[b], gsem[b])

    def block(blk, carry):
      j0 = blk * nb
      for b in range(nb):
        j = j0 + b
        pltpu.make_async_copy(u_hbm.at[gv.at[b]], rows_v[b], gsem[b]).wait()
        pltpu.sync_copy(rows_v[b], acc.at[sv.at[j]], add=True)

        @pl.when(j + nb < cpw)
        def _():
          pltpu.async_copy(u_hbm.at[gv.at[j + nb]], rows_v[b], gsem[b])

      return carry

    lax.fori_loop(0, cpw // nb, block, 0)
    plsc.subcore_barrier()

    @pl.when(s < _NS - 1)
    def _():
      r0 = s * rpw
      pltpu.sync_copy(acc.at[pl.ds(r0, rpw)], out_hbm.at[c].at[pl.ds(r0, rpw)])

    @pl.when(s == _NS - 1)
    def _():
      r0 = (_NS - 1) * rpw
      pltpu.sync_copy(acc.at[pl.ds(r0, sz_last)],
                      out_hbm.at[c].at[pl.ds(r0, sz_last)])

  return seg


@functools.cache
def _degree(n_rows: int, feat: int, e_pad: int):
  cb = _CB
  rpw = -(-(n_rows + 1) // (_NS * 8)) * 8
  npad = rpw * _NS
  cpw = e_pad // (_NW * cb)
  sz_last = n_rows - (_NS - 1) * rpw

  @functools.partial(
      pl.kernel, mesh=_mesh(),
      out_type=jax.ShapeDtypeStruct((_NC, n_rows, feat), jnp.float32),
      compiler_params=pltpu.CompilerParams(use_tc_tiling_on_sc=False),
      scratch_types=[
          pltpu.VMEM((cpw, cb), jnp.int32),
          pltpu.VMEM((cb, feat), jnp.float32),
          pltpu.VMEM_SHARED((npad, feat), jnp.float32),
      ],
  )
  def deg(sidx_hbm, ones_hbm, zeros_hbm, out_hbm, sv, ones_v, acc):
    c = lax.axis_index("c")
    s = lax.axis_index("s")
    w = c * _NS + s
    pltpu.sync_copy(ones_hbm, ones_v)
    pltpu.sync_copy(zeros_hbm, acc.at[pl.ds(s * rpw, rpw)])
    cr0 = pl.multiple_of(w * cpw, 8)
    pltpu.sync_copy(sidx_hbm.at[pl.ds(cr0, cpw)], sv)
    plsc.subcore_barrier()

    def chunk(j, carry):
      pltpu.sync_copy(ones_v, acc.at[sv.at[j]], add=True)
      return carry

    lax.fori_loop(0, cpw, chunk, 0)
    plsc.subcore_barrier()

    @pl.when(s < _NS - 1)
    def _():
      r0 = s * rpw
      pltpu.sync_copy(acc.at[pl.ds(r0, rpw)], out_hbm.at[c].at[pl.ds(r0, rpw)])

    @pl.when(s == _NS - 1)
    def _():
      r0 = (_NS - 1) * rpw
      pltpu.sync_copy(acc.at[pl.ds(r0, sz_last)],
                      out_hbm.at[c].at[pl.ds(r0, sz_last)])

  return deg


@functools.cache
def _gather2(n_rows: int, feat: int, e_pad: int, tc_tiling: bool = True,
             dtype=jnp.float32):
  cb = _CB
  cpw = e_pad // (_NW * cb)

  @functools.partial(
      pl.kernel, mesh=_mesh(),
      out_type=(jax.ShapeDtypeStruct((e_pad, feat), dtype),
                jax.ShapeDtypeStruct((e_pad, feat), dtype)),
      compiler_params=pltpu.CompilerParams(use_tc_tiling_on_sc=tc_tiling),
      scratch_types=[
          pltpu.VMEM((cpw, cb), jnp.int32),
          pltpu.VMEM((cpw, cb), jnp.int32),
          [pltpu.VMEM((cb, feat), dtype)] * 2,
          [pltpu.VMEM((cb, feat), dtype)] * 2,
          [pltpu.SemaphoreType.DMA] * 2,
          [pltpu.SemaphoreType.DMA] * 2,
      ],
  )
  def gat(tab_hbm, aidx_hbm, bidx_hbm, oa_hbm, ob_hbm,
          av, bv, rows_a, rows_b, gsa, gsb):
    c = lax.axis_index("c")
    s = lax.axis_index("s")
    w = c * _NS + s
    w_base = pl.multiple_of(w * (cpw * cb), 8)
    cr0 = pl.multiple_of(w * cpw, 8)
    pltpu.sync_copy(aidx_hbm.at[pl.ds(cr0, cpw)], av)
    pltpu.sync_copy(bidx_hbm.at[pl.ds(cr0, cpw)], bv)
    for b in range(2):
      pltpu.async_copy(tab_hbm.at[av.at[b]], rows_a[b], gsa[b])
      pltpu.async_copy(tab_hbm.at[bv.at[b]], rows_b[b], gsb[b])

    def block(blk, carry):
      j0 = blk * 2
      for b in range(2):
        j = j0 + b
        base = pl.multiple_of(w_base + j * cb, 8)
        pltpu.make_async_copy(tab_hbm.at[av.at[b]], rows_a[b], gsa[b]).wait()
        pltpu.sync_copy(rows_a[b], oa_hbm.at[pl.ds(base, cb)])
        pltpu.make_async_copy(tab_hbm.at[bv.at[b]], rows_b[b], gsb[b]).wait()
        pltpu.sync_copy(rows_b[b], ob_hbm.at[pl.ds(base, cb)])

        @pl.when(j + 2 < cpw)
        def _():
          pltpu.async_copy(tab_hbm.at[av.at[j + 2]], rows_a[b], gsa[b])
          pltpu.async_copy(tab_hbm.at[bv.at[j + 2]], rows_b[b], gsb[b])

      return carry

    lax.fori_loop(0, cpw // 2, block, 0)

  return gat


def _vspec(bn, *trail):
  return pl.BlockSpec((bn,) + trail, lambda i: (i,) + (0,) * len(trail))


def _wspec(shape):
  return pl.BlockSpec(shape, lambda i: (0,) * len(shape))


def _k_prescale(degp, x_seq, bn):
  t, n, f = x_seq.shape
  fd = degp.shape[2]

  def body(dp_ref, x_ref, dv_ref, xs_ref):
    deg = dp_ref[0, :, 0] + dp_ref[1, :, 0]
    dv = jnp.where(deg > 0, 1.0 / jnp.sqrt(jnp.maximum(deg, 1e-12)), 0.0)
    dv_ref[...] = dv[:, None]
    xs_ref[...] = (x_ref[...] * dv[None, :, None]).astype(xs_ref.dtype)

  return pl.pallas_call(
      body,
      grid=(n // bn,),
      in_specs=[pl.BlockSpec((2, bn, fd), lambda i: (0, i, 0)),
                pl.BlockSpec((t, bn, f), lambda i: (0, i, 0))],
      out_specs=[_vspec(bn, 1), pl.BlockSpec((t, bn, f), lambda i: (0, i, 0))],
      out_shape=[jax.ShapeDtypeStruct((n, 1), jnp.float32),
                 jax.ShapeDtypeStruct((t, n, f), jnp.bfloat16)],
  )(degp, x_seq)


def _k_mid(p1, dinv, bn, m1_dtype=jnp.float32):
  _, n, f = p1.shape

  def body(p_ref, dv_ref, lap_ref, m1_ref):
    ps = p_ref[0].astype(jnp.float32) + p_ref[1].astype(jnp.float32)
    dv = dv_ref[...]
    lap_ref[...] = -dv * ps
    m1_ref[...] = ((dv * dv) * ps).astype(m1_ref.dtype)

  return pl.pallas_call(
      body,
      grid=(n // bn,),
      in_specs=[pl.BlockSpec((2, bn, f), lambda i: (0, i, 0)), _vspec(bn, 1)],
      out_specs=[_vspec(bn, f), _vspec(bn, f)],
      out_shape=[jax.ShapeDtypeStruct((n, f), jnp.float32),
                 jax.ShapeDtypeStruct((n, f), m1_dtype)],
  )(p1, dinv)


def _k_cheb(v, lapv, p2, dinv, a0m2, a1, a2x2, bias, bn, t_idx):
  _, n, f = v.shape
  v_spec = pl.BlockSpec((1, bn, f), lambda i, t=t_idx: (t, i, 0))
  fo = a1.shape[1]

  def body(v_ref, lap_ref, p2_ref, dv_ref, a0_ref, a1_ref, a2_ref, b_ref,
           o_ref):
    vv = v_ref[...].reshape(bn, f)
    dv = dv_ref[...]
    l2 = dv * (p2_ref[0].astype(jnp.float32) + p2_ref[1].astype(jnp.float32))
    acc = jnp.dot(vv, a0_ref[...], preferred_element_type=jnp.float32)
    acc += jnp.dot(lap_ref[...], a1_ref[...], preferred_element_type=jnp.float32)
    acc += jnp.dot(l2, a2_ref[...], preferred_element_type=jnp.float32)
    o_ref[...] = acc + b_ref[...]

  return pl.pallas_call(
      body,
      grid=(n // bn,),
      in_specs=[v_spec, _vspec(bn, f),
                pl.BlockSpec((2, bn, f), lambda i: (0, i, 0)), _vspec(bn, 1),
                _wspec(a0m2.shape), _wspec(a1.shape), _wspec(a2x2.shape),
                _wspec(bias.shape)],
      out_specs=_vspec(bn, fo),
      out_shape=jax.ShapeDtypeStruct((n, fo), jnp.float32),
  )(v, lapv, p2, dinv, a0m2, a1, a2x2, bias)


def _k_t0(cx0, dinv, b_hz, b_hn, hg, bn):
  n = cx0.shape[0]

  def body(cx_ref, dv_ref, bz_ref, bnn_ref, h_ref, u_ref):
    cx = cx_ref[...]
    z = jax.nn.sigmoid(cx[:, :hg] + bz_ref[...])
    htil = jnp.tanh(cx[:, 2 * hg:] + bnn_ref[...])
    h1 = (1.0 - z) * htil
    h_ref[...] = h1
    u_ref[...] = (dv_ref[...] * h1).astype(u_ref.dtype)

  return pl.pallas_call(
      body,
      grid=(n // bn,),
      in_specs=[_vspec(bn, 3 * hg), _vspec(bn, 1), _wspec(b_hz.shape),
                _wspec(b_hn.shape)],
      out_specs=[_vspec(bn, hg), _vspec(bn, hg)],
      out_shape=[jax.ShapeDtypeStruct((n, hg), jnp.float32),
                 jax.ShapeDtypeStruct((n, hg), jnp.bfloat16)],
  )(cx0, dinv, b_hz, b_hn)


def _k_zr(cx, h, laph, p2h, dinv, b0m2, b1w, b2x2, bzr, hg, bn):
  n = cx.shape[0]

  def body(cx_ref, h_ref, lap_ref, p2_ref, dv_ref, b0_ref, b1_ref, b2_ref,
           bb_ref, z_ref, hr_ref, u_ref):
    dv = dv_ref[...]
    hh = h_ref[...]
    l2 = dv * (p2_ref[0].astype(jnp.float32) + p2_ref[1].astype(jnp.float32))
    ch = jnp.dot(hh, b0_ref[...], preferred_element_type=jnp.float32)
    ch += jnp.dot(lap_ref[...], b1_ref[...],
                  preferred_element_type=jnp.float32)
    ch += jnp.dot(l2, b2_ref[...], preferred_element_type=jnp.float32)
    ch += bb_ref[...]
    cxv = cx_ref[...]
    z = jax.nn.sigmoid(cxv[:, :hg] + ch[:, :hg])
    r = jax.nn.sigmoid(cxv[:, hg:2 * hg] + ch[:, hg:])
    hr = hh * r
    z_ref[...] = z
    hr_ref[...] = hr
    u_ref[...] = (dv * hr).astype(u_ref.dtype)

  return pl.pallas_call(
      body,
      grid=(n // bn,),
      in_specs=[_vspec(bn, 3 * hg), _vspec(bn, hg), _vspec(bn, hg),
                pl.BlockSpec((2, bn, hg), lambda i: (0, i, 0)), _vspec(bn, 1),
                _wspec(b0m2.shape), _wspec(b1w.shape), _wspec(b2x2.shape),
                _wspec(bzr.shape)],
      out_specs=[_vspec(bn, hg), _vspec(bn, hg), _vspec(bn, hg)],
      out_shape=[jax.ShapeDtypeStruct((n, hg), jnp.float32),
                 jax.ShapeDtypeStruct((n, hg), jnp.float32),
                 jax.ShapeDtypeStruct((n, hg), jnp.bfloat16)],
  )(cx, h, laph, p2h, dinv, b0m2, b1w, b2x2, bzr)


def _k_upd(cx, h, z, hr, lapn, p2n, dinv, c0m2, c1w, c2x2, bnn, hg, bn,
           hn_dtype=jnp.float32):
  n = cx.shape[0]

  def body(cx_ref, h_ref, z_ref, hr_ref, lap_ref, p2_ref, dv_ref, c0_ref,
           c1_ref, c2_ref, bb_ref, hn_ref, u_ref):
    dv = dv_ref[...]
    l2 = dv * (p2_ref[0].astype(jnp.float32) + p2_ref[1].astype(jnp.float32))
    ch = jnp.dot(hr_ref[...], c0_ref[...], preferred_element_type=jnp.float32)
    ch += jnp.dot(lap_ref[...], c1_ref[...],
                  preferred_element_type=jnp.float32)
    ch += jnp.dot(l2, c2_ref[...], preferred_element_type=jnp.float32)
    htil = jnp.tanh(cx_ref[...][:, 2 * hg:] + ch + bb_ref[...])
    z = z_ref[...]
    hn = z * h_ref[...] + (1.0 - z) * htil
    hn_ref[...] = hn.astype(hn_ref.dtype)
    u_ref[...] = (dv * hn).astype(u_ref.dtype)

  return pl.pallas_call(
      body,
      grid=(n // bn,),
      in_specs=[_vspec(bn, 3 * hg), _vspec(bn, hg), _vspec(bn, hg),
                _vspec(bn, hg), _vspec(bn, hg),
                pl.BlockSpec((2, bn, hg), lambda i: (0, i, 0)), _vspec(bn, 1),
                _wspec(c0m2.shape), _wspec(c1w.shape), _wspec(c2x2.shape),
                _wspec(bnn.shape)],
      out_specs=[_vspec(bn, hg), _vspec(bn, hg)],
      out_shape=[jax.ShapeDtypeStruct((n, hg), hn_dtype),
                 jax.ShapeDtypeStruct((n, hg), jnp.bfloat16)],
  )(cx, h, z, hr, lapn, p2n, dinv, c0m2, c1w, c2x2, bnn)


def _k_decoder(hs, hd, tc16, st, w_s, w_d, w_tblk, w_st, b_ih, w_hh, b_hh,
               w1, b1, w2, b2, e, t_steps, hg, hdec, be):
  hp = hs.shape[1]
  fs = w_st.shape[0]
  ftt = tc16.shape[1]

  def body(hs_ref, hd_ref, tc_ref, st_ref, ws_ref, wd_ref, wt_ref, wst_ref,
           bih_ref, whh_ref, bhh_ref, w1_ref, b1_ref, w2_ref, b2_ref, o_ref):
    gi_base = jnp.dot(hs_ref[...], ws_ref[...],
                      preferred_element_type=jnp.float32)
    gi_base += jnp.dot(hd_ref[...], wd_ref[...],
                       preferred_element_type=jnp.float32)
    gi_base += jnp.dot(st_ref[...], wst_ref[...],
                       preferred_element_type=jnp.float32)
    gi_base += bih_ref[...]
    gt_all = jnp.dot(tc_ref[...], wt_ref[...],
                     preferred_element_type=jnp.float32)
    h = jnp.zeros((be, hdec), jnp.float32)
    cols = []
    for t in range(t_steps):
      gi = gi_base + gt_all[:, 3 * hdec * t:3 * hdec * (t + 1)]
      gh = jnp.dot(h, whh_ref[...], preferred_element_type=jnp.float32)
      gh += bhh_ref[...]
      r = jax.nn.sigmoid(gi[:, :hdec] + gh[:, :hdec])
      z = jax.nn.sigmoid(gi[:, hdec:2 * hdec] + gh[:, hdec:2 * hdec])
      nn = jnp.tanh(gi[:, 2 * hdec:] + r * gh[:, 2 * hdec:])
      h = (1.0 - z) * nn + z * h
      hid = jax.nn.relu(
          jnp.dot(h, w1_ref[...], preferred_element_type=jnp.float32)
          + b1_ref[...])
      cols.append(jnp.dot(hid, w2_ref[...], preferred_element_type=jnp.float32)
                  + b2_ref[...])
    o_ref[...] = jnp.concatenate(cols, axis=1)

  return pl.pallas_call(
      body,
      grid=(e // be,),
      in_specs=[_vspec(be, hp), _vspec(be, hp), _vspec(be, ftt),
                _vspec(be, fs),
                _wspec(w_s.shape), _wspec(w_d.shape), _wspec(w_tblk.shape),
                _wspec(w_st.shape), _wspec(b_ih.shape), _wspec(w_hh.shape),
                _wspec(b_hh.shape), _wspec(w1.shape), _wspec(b1.shape),
                _wspec(w2.shape), _wspec(b2.shape)],
      out_specs=_vspec(be, t_steps),
      out_shape=jax.ShapeDtypeStruct((e, t_steps), jnp.float32),
  )(hs, hd, tc16, st, w_s, w_d, w_tblk, w_st, b_ih, w_hh, b_hh, w1, b1, w2, b2)


def kernel(x_seq, edge_index, time_seq, static_feats,
           W_xz, b_xz, W_hz, b_hz, W_xr, b_xr, W_hr, b_hr,
           W_xh, b_xh, W_hn, b_hn,
           W_ih, W_hh, b_ih, b_hh, W1, b1, W2, b2):
  t_steps, n, f_node = x_seq.shape
  e = edge_index.shape[1]
  hg = W_hz.shape[2]
  hdec = W_hh.shape[0] // 3
  ft = time_seq.shape[2]

  cpw = -(-e // (_NW * _CB))
  e_pad = _NW * cpw * _CB
  pad = e_pad - e
  bn = 2000
  be = 2000

  src = edge_index[0]
  dst = edge_index[1]
  zpad_i = jnp.zeros((pad,), jnp.int32)
  npad_i = jnp.full((pad,), n, jnp.int32)
  src_g = jnp.concatenate([src, zpad_i]).reshape(-1, _CB)
  dst_g = jnp.concatenate([dst, zpad_i]).reshape(-1, _CB)
  src_s = jnp.concatenate([src, npad_i]).reshape(-1, _CB)
  dst_s = jnp.concatenate([dst, npad_i]).reshape(-1, _CB)

  rpw = -(-(n + 1) // (_NS * 8)) * 8
  z_fn = jnp.zeros((rpw, f_node), jnp.bfloat16)
  z_hg = jnp.zeros((rpw, hg), jnp.bfloat16)
  ones_row = jnp.ones((_CB, 16), jnp.float32)

  seg_fn = _segsum(n, f_node, e_pad, tc_tiling=False, dtype=jnp.bfloat16,
                   nb=4)
  seg_hg = _segsum(n, hg, e_pad, tc_tiling=False,
                   dtype=jnp.bfloat16, nb=4)
  deg_k = _degree(n, 16, e_pad)
  gat_k = _gather2(n, hg, e_pad, tc_tiling=False,
                   dtype=jnp.bfloat16)

  degp = deg_k(src_s, ones_row, jnp.zeros((rpw, 16), jnp.float32))
  dinv, xs = _k_prescale(degp, x_seq, bn)

  a_k = [jnp.concatenate([W_xz[k], W_xr[k], W_xh[k]], axis=1) for k in range(3)]
  a0m2, a1w, a2x2 = a_k[0] - a_k[2], a_k[1], 2.0 * a_k[2]
  bx = jnp.concatenate([b_xz, b_xr, b_xh]).reshape(1, 3 * hg)
  b_k = [jnp.concatenate([W_hz[k], W_hr[k]], axis=1) for k in range(3)]
  b0m2, b1w, b2x2 = b_k[0] - b_k[2], b_k[1], 2.0 * b_k[2]
  bzr = jnp.concatenate([b_hz, b_hr]).reshape(1, 2 * hg)
  c0m2, c1w, c2x2 = W_hn[0] - W_hn[2], W_hn[1], 2.0 * W_hn[2]
  bz2 = b_hz.reshape(1, hg)
  bn2 = b_hn.reshape(1, hg)

  cxs = []
  for t in range(t_steps):
    p1 = seg_fn(xs[t], src_g, dst_s, z_fn)
    lapx, m1x = _k_mid(p1, dinv, bn, m1_dtype=jnp.bfloat16)
    p2 = seg_fn(m1x, src_g, dst_s, z_fn)
    cxs.append(_k_cheb(x_seq, lapx, p2, dinv, a0m2, a1w, a2x2, bx, bn, t))

  h_cur, u_cur = _k_t0(cxs[0], dinv, bz2, bn2, hg, bn)
  for t in range(1, t_steps):
    p1h = seg_hg(u_cur, src_g, dst_s, z_hg)
    laph, m1h = _k_mid(p1h, dinv, bn, m1_dtype=jnp.bfloat16)
    p2h = seg_hg(m1h, src_g, dst_s, z_hg)
    z_gate, hr, uhr = _k_zr(cxs[t], h_cur, laph, p2h, dinv, b0m2, b1w, b2x2,
                            bzr, hg, bn)
    p1n = seg_hg(uhr, src_g, dst_s, z_hg)
    lapn, m1n = _k_mid(p1n, dinv, bn, m1_dtype=jnp.bfloat16)
    p2n = seg_hg(m1n, src_g, dst_s, z_hg)
    h_cur, u_cur = _k_upd(cxs[t], h_cur, z_gate, hr, lapn, p2n, dinv, c0m2,
                          c1w, c2x2, bn2, hg, bn,
                          hn_dtype=(jnp.bfloat16 if t == t_steps - 1
                                    else jnp.float32))

  hs, hd = gat_k(h_cur, src_g, dst_g)
  tc16 = jnp.transpose(time_seq, (1, 0, 2)).reshape(e, t_steps * ft)
  w_s = W_ih[:, :hg].T
  w_d = W_ih[:, hg:2 * hg].T
  w_t = W_ih[:, 2 * hg:2 * hg + ft].T
  w_st = W_ih[:, 2 * hg + ft:].T
  w_tblk = jnp.zeros((t_steps * ft, t_steps * 3 * hdec), jnp.float32)
  for t in range(t_steps):
    w_tblk = w_tblk.at[ft * t:ft * (t + 1),
                       3 * hdec * t:3 * hdec * (t + 1)].set(w_t)
  out_et = _k_decoder(hs, hd, tc16, static_feats,
                      w_s, w_d, w_tblk, w_st, b_ih.reshape(1, -1),
                      W_hh.T, b_hh.reshape(1, -1),
                      W1, b1.reshape(1, -1), W2, b2.reshape(1, -1),
                      e, t_steps, hg, hdec, be)
  return out_et.T


# speedup vs baseline: 2.2899x; 1.0004x over previous
import functools

import jax
import jax.numpy as jnp
from jax import lax
from jax.experimental import pallas as pl
from jax.experimental.pallas import tpu as pltpu
from jax.experimental.pallas import tpu_sc as plsc

_NC, _NS = 2, 16
_NW = _NC * _NS
_CB = 128


def _mesh():
  return plsc.VectorSubcoreMesh(core_axis_name="c", subcore_axis_name="s",
                                num_cores=_NC, num_subcores=_NS)


_NB = 2


@functools.cache
def _segsum(n_rows: int, feat: int, e_pad: int, tc_tiling: bool = True,
            dtype=jnp.float32, nb: int = _NB):
  cb = _CB
  rpw = -(-(n_rows + 1) // (_NS * 8)) * 8
  npad = rpw * _NS
  cpw = e_pad // (_NW * cb)
  assert cpw % nb == 0 and cpw > nb
  sz_last = n_rows - (_NS - 1) * rpw

  @functools.partial(
      pl.kernel, mesh=_mesh(),
      out_type=jax.ShapeDtypeStruct((_NC, n_rows, feat), dtype),
      compiler_params=pltpu.CompilerParams(use_tc_tiling_on_sc=tc_tiling),
      scratch_types=[
          pltpu.VMEM((cpw, cb), jnp.int32),
          pltpu.VMEM((cpw, cb), jnp.int32),
          [pltpu.VMEM((cb, feat), dtype)] * nb,
          pltpu.VMEM_SHARED((npad, feat), dtype),
          [pltpu.SemaphoreType.DMA] * nb,
      ],
  )
  def seg(u_hbm, gidx_hbm, sidx_hbm, zeros_hbm, out_hbm,
          gv, sv, rows_v, acc, gsem):
    c = lax.axis_index("c")
    s = lax.axis_index("s")
    w = c * _NS + s
    pltpu.sync_copy(zeros_hbm, acc.at[pl.ds(s * rpw, rpw)])
    cr0 = pl.multiple_of(w * cpw, 8)
    pltpu.sync_copy(gidx_hbm.at[pl.ds(cr0, cpw)], gv)
    pltpu.sync_copy(sidx_hbm.at[pl.ds(cr0, cpw)], sv)
    plsc.subcore_barrier()

    for b in range(nb):
      pltpu.async_copy(u_hbm.at[gv.at[b]], rows_v[b], gsem[b])

    def block(blk, carry):
      j0 = blk * nb
      for b in range(nb):
        j = j0 + b
        pltpu.make_async_copy(u_hbm.at[gv.at[b]], rows_v[b], gsem[b]).wait()
        pltpu.sync_copy(rows_v[b], acc.at[sv.at[j]], add=True)

        @pl.when(j + nb < cpw)
        def _():
          pltpu.async_copy(u_hbm.at[gv.at[j + nb]], rows_v[b], gsem[b])

      return carry

    lax.fori_loop(0, cpw // nb, block, 0)
    plsc.subcore_barrier()

    @pl.when(s < _NS - 1)
    def _():
      r0 = s * rpw
      pltpu.sync_copy(acc.at[pl.ds(r0, rpw)], out_hbm.at[c].at[pl.ds(r0, rpw)])

    @pl.when(s == _NS - 1)
    def _():
      r0 = (_NS - 1) * rpw
      pltpu.sync_copy(acc.at[pl.ds(r0, sz_last)],
                      out_hbm.at[c].at[pl.ds(r0, sz_last)])

  return seg


@functools.cache
def _degree(n_rows: int, feat: int, e_pad: int):
  cb = _CB
  rpw = -(-(n_rows + 1) // (_NS * 8)) * 8
  npad = rpw * _NS
  cpw = e_pad // (_NW * cb)
  sz_last = n_rows - (_NS - 1) * rpw

  @functools.partial(
      pl.kernel, mesh=_mesh(),
      out_type=jax.ShapeDtypeStruct((_NC, n_rows, feat), jnp.float32),
      compiler_params=pltpu.CompilerParams(use_tc_tiling_on_sc=False),
      scratch_types=[
          pltpu.VMEM((cpw, cb), jnp.int32),
          pltpu.VMEM((cb, feat), jnp.float32),
          pltpu.VMEM_SHARED((npad, feat), jnp.float32),
      ],
  )
  def deg(sidx_hbm, ones_hbm, zeros_hbm, out_hbm, sv, ones_v, acc):
    c = lax.axis_index("c")
    s = lax.axis_index("s")
    w = c * _NS + s
    pltpu.sync_copy(ones_hbm, ones_v)
    pltpu.sync_copy(zeros_hbm, acc.at[pl.ds(s * rpw, rpw)])
    cr0 = pl.multiple_of(w * cpw, 8)
    pltpu.sync_copy(sidx_hbm.at[pl.ds(cr0, cpw)], sv)
    plsc.subcore_barrier()

    def chunk(j, carry):
      pltpu.sync_copy(ones_v, acc.at[sv.at[j]], add=True)
      return carry

    lax.fori_loop(0, cpw, chunk, 0)
    plsc.subcore_barrier()

    @pl.when(s < _NS - 1)
    def _():
      r0 = s * rpw
      pltpu.sync_copy(acc.at[pl.ds(r0, rpw)], out_hbm.at[c].at[pl.ds(r0, rpw)])

    @pl.when(s == _NS - 1)
    def _():
      r0 = (_NS - 1) * rpw
      pltpu.sync_copy(acc.at[pl.ds(r0, sz_last)],
                      out_hbm.at[c].at[pl.ds(r0, sz_last)])

  return deg


@functools.cache
def _gather2(n_rows: int, feat: int, e_pad: int, tc_tiling: bool = True,
             dtype=jnp.float32):
  cb = _CB
  cpw = e_pad // (_NW * cb)

  @functools.partial(
      pl.kernel, mesh=_mesh(),
      out_type=(jax.ShapeDtypeStruct((e_pad, feat), dtype),
                jax.ShapeDtypeStruct((e_pad, feat), dtype)),
      compiler_params=pltpu.CompilerParams(use_tc_tiling_on_sc=tc_tiling),
      scratch_types=[
          pltpu.VMEM((cpw, cb), jnp.int32),
          pltpu.VMEM((cpw, cb), jnp.int32),
          [pltpu.VMEM((cb, feat), dtype)] * 2,
          [pltpu.VMEM((cb, feat), dtype)] * 2,
          [pltpu.SemaphoreType.DMA] * 2,
          [pltpu.SemaphoreType.DMA] * 2,
      ],
  )
  def gat(tab_hbm, aidx_hbm, bidx_hbm, oa_hbm, ob_hbm,
          av, bv, rows_a, rows_b, gsa, gsb):
    c = lax.axis_index("c")
    s = lax.axis_index("s")
    w = c * _NS + s
    w_base = pl.multiple_of(w * (cpw * cb), 8)
    cr0 = pl.multiple_of(w * cpw, 8)
    pltpu.sync_copy(aidx_hbm.at[pl.ds(cr0, cpw)], av)
    pltpu.sync_copy(bidx_hbm.at[pl.ds(cr0, cpw)], bv)
    for b in range(2):
      pltpu.async_copy(tab_hbm.at[av.at[b]], rows_a[b], gsa[b])
      pltpu.async_copy(tab_hbm.at[bv.at[b]], rows_b[b], gsb[b])

    def block(blk, carry):
      j0 = blk * 2
      for b in range(2):
        j = j0 + b
        base = pl.multiple_of(w_base + j * cb, 8)
        pltpu.make_async_copy(tab_hbm.at[av.at[b]], rows_a[b], gsa[b]).wait()
        pltpu.sync_copy(rows_a[b], oa_hbm.at[pl.ds(base, cb)])
        pltpu.make_async_copy(tab_hbm.at[bv.at[b]], rows_b[b], gsb[b]).wait()
        pltpu.sync_copy(rows_b[b], ob_hbm.at[pl.ds(base, cb)])

        @pl.when(j + 2 < cpw)
        def _():
          pltpu.async_copy(tab_hbm.at[av.at[j + 2]], rows_a[b], gsa[b])
          pltpu.async_copy(tab_hbm.at[bv.at[j + 2]], rows_b[b], gsb[b])

      return carry

    lax.fori_loop(0, cpw // 2, block, 0)

  return gat


def _vspec(bn, *trail):
  return pl.BlockSpec((bn,) + trail, lambda i: (i,) + (0,) * len(trail))


def _wspec(shape):
  return pl.BlockSpec(shape, lambda i: (0,) * len(shape))


def _k_prescale(degp, x_seq, bn):
  t, n, f = x_seq.shape
  fd = degp.shape[2]

  def body(dp_ref, x_ref, dv_ref, xs_ref):
    deg = dp_ref[0, :, 0] + dp_ref[1, :, 0]
    dv = jnp.where(deg > 0, 1.0 / jnp.sqrt(jnp.maximum(deg, 1e-12)), 0.0)
    dv_ref[...] = dv[:, None]
    xs_ref[...] = (x_ref[...] * dv[None, :, None]).astype(xs_ref.dtype)

  return pl.pallas_call(
      body,
      grid=(n // bn,),
      in_specs=[pl.BlockSpec((2, bn, fd), lambda i: (0, i, 0)),
                pl.BlockSpec((t, bn, f), lambda i: (0, i, 0))],
      out_specs=[_vspec(bn, 1), pl.BlockSpec((t, bn, f), lambda i: (0, i, 0))],
      out_shape=[jax.ShapeDtypeStruct((n, 1), jnp.float32),
                 jax.ShapeDtypeStruct((t, n, f), jnp.bfloat16)],
  )(degp, x_seq)


def _k_mid(p1, dinv, bn, m1_dtype=jnp.float32):
  _, n, f = p1.shape

  def body(p_ref, dv_ref, lap_ref, m1_ref):
    ps = p_ref[0].astype(jnp.float32) + p_ref[1].astype(jnp.float32)
    dv = dv_ref[...]
    lap_ref[...] = -dv * ps
    m1_ref[...] = ((dv * dv) * ps).astype(m1_ref.dtype)

  return pl.pallas_call(
      body,
      grid=(n // bn,),
      in_specs=[pl.BlockSpec((2, bn, f), lambda i: (0, i, 0)), _vspec(bn, 1)],
      out_specs=[_vspec(bn, f), _vspec(bn, f)],
      out_shape=[jax.ShapeDtypeStruct((n, f), jnp.float32),
                 jax.ShapeDtypeStruct((n, f), m1_dtype)],
  )(p1, dinv)


def _k_cheb(v, lapv, p2, dinv, a0m2, a1, a2x2, bias, bn, t_idx):
  _, n, f = v.shape
  v_spec = pl.BlockSpec((1, bn, f), lambda i, t=t_idx: (t, i, 0))
  fo = a1.shape[1]

  def body(v_ref, lap_ref, p2_ref, dv_ref, a0_ref, a1_ref, a2_ref, b_ref,
           o_ref):
    vv = v_ref[...].reshape(bn, f)
    dv = dv_ref[...]
    l2 = dv * (p2_ref[0].astype(jnp.float32) + p2_ref[1].astype(jnp.float32))
    acc = jnp.dot(vv, a0_ref[...], preferred_element_type=jnp.float32)
    acc += jnp.dot(lap_ref[...], a1_ref[...], preferred_element_type=jnp.float32)
    acc += jnp.dot(l2, a2_ref[...], preferred_element_type=jnp.float32)
    o_ref[...] = acc + b_ref[...]

  return pl.pallas_call(
      body,
      grid=(n // bn,),
      in_specs=[v_spec, _vspec(bn, f),
                pl.BlockSpec((2, bn, f), lambda i: (0, i, 0)), _vspec(bn, 1),
                _wspec(a0m2.shape), _wspec(a1.shape), _wspec(a2x2.shape),
                _wspec(bias.shape)],
      out_specs=_vspec(bn, fo),
      out_shape=jax.ShapeDtypeStruct((n, fo), jnp.float32),
  )(v, lapv, p2, dinv, a0m2, a1, a2x2, bias)


def _k_t0(cx0, dinv, b_hz, b_hn, hg, bn):
  n = cx0.shape[0]

  def body(cx_ref, dv_ref, bz_ref, bnn_ref, h_ref, u_ref):
    cx = cx_ref[...]
    z = jax.nn.sigmoid(cx[:, :hg] + bz_ref[...])
    htil = jnp.tanh(cx[:, 2 * hg:] + bnn_ref[...])
    h1 = (1.0 - z) * htil
    h_ref[...] = h1
    u_ref[...] = (dv_ref[...] * h1).astype(u_ref.dtype)

  return pl.pallas_call(
      body,
      grid=(n // bn,),
      in_specs=[_vspec(bn, 3 * hg), _vspec(bn, 1), _wspec(b_hz.shape),
                _wspec(b_hn.shape)],
      out_specs=[_vspec(bn, hg), _vspec(bn, hg)],
      out_shape=[jax.ShapeDtypeStruct((n, hg), jnp.float32),
                 jax.ShapeDtypeStruct((n, hg), jnp.bfloat16)],
  )(cx0, dinv, b_hz, b_hn)


def _k_zr(cx, h, laph, p2h, dinv, b0m2, b1w, b2x2, bzr, hg, bn):
  n = cx.shape[0]

  def body(cx_ref, h_ref, lap_ref, p2_ref, dv_ref, b0_ref, b1_ref, b2_ref,
           bb_ref, z_ref, hr_ref, u_ref):
    dv = dv_ref[...]
    hh = h_ref[...]
    l2 = dv * (p2_ref[0].astype(jnp.float32) + p2_ref[1].astype(jnp.float32))
    ch = jnp.dot(hh, b0_ref[...], preferred_element_type=jnp.float32)
    ch += jnp.dot(lap_ref[...], b1_ref[...],
                  preferred_element_type=jnp.float32)
    ch += jnp.dot(l2, b2_ref[...], preferred_element_type=jnp.float32)
    ch += bb_ref[...]
    cxv = cx_ref[...]
    z = jax.nn.sigmoid(cxv[:, :hg] + ch[:, :hg])
    r = jax.nn.sigmoid(cxv[:, hg:2 * hg] + ch[:, hg:])
    hr = hh * r
    z_ref[...] = z
    hr_ref[...] = hr
    u_ref[...] = (dv * hr).astype(u_ref.dtype)

  return pl.pallas_call(
      body,
      grid=(n // bn,),
      in_specs=[_vspec(bn, 3 * hg), _vspec(bn, hg), _vspec(bn, hg),
                pl.BlockSpec((2, bn, hg), lambda i: (0, i, 0)), _vspec(bn, 1),
                _wspec(b0m2.shape), _wspec(b1w.shape), _wspec(b2x2.shape),
                _wspec(bzr.shape)],
      out_specs=[_vspec(bn, hg), _vspec(bn, hg), _vspec(bn, hg)],
      out_shape=[jax.ShapeDtypeStruct((n, hg), jnp.float32),
                 jax.ShapeDtypeStruct((n, hg), jnp.float32),
                 jax.ShapeDtypeStruct((n, hg), jnp.bfloat16)],
  )(cx, h, laph, p2h, dinv, b0m2, b1w, b2x2, bzr)


def _k_upd(cx, h, z, hr, lapn, p2n, dinv, c0m2, c1w, c2x2, bnn, hg, bn,
           hn_dtype=jnp.float32):
  n = cx.shape[0]

  def body(cx_ref, h_ref, z_ref, hr_ref, lap_ref, p2_ref, dv_ref, c0_ref,
           c1_ref, c2_ref, bb_ref, hn_ref, u_ref):
    dv = dv_ref[...]
    l2 = dv * (p2_ref[0].astype(jnp.float32) + p2_ref[1].astype(jnp.float32))
    ch = jnp.dot(hr_ref[...], c0_ref[...], preferred_element_type=jnp.float32)
    ch += jnp.dot(lap_ref[...], c1_ref[...],
                  preferred_element_type=jnp.float32)
    ch += jnp.dot(l2, c2_ref[...], preferred_element_type=jnp.float32)
    htil = jnp.tanh(cx_ref[...][:, 2 * hg:] + ch + bb_ref[...])
    z = z_ref[...]
    hn = z * h_ref[...] + (1.0 - z) * htil
    hn_ref[...] = hn.astype(hn_ref.dtype)
    u_ref[...] = (dv * hn).astype(u_ref.dtype)

  return pl.pallas_call(
      body,
      grid=(n // bn,),
      in_specs=[_vspec(bn, 3 * hg), _vspec(bn, hg), _vspec(bn, hg),
                _vspec(bn, hg), _vspec(bn, hg),
                pl.BlockSpec((2, bn, hg), lambda i: (0, i, 0)), _vspec(bn, 1),
                _wspec(c0m2.shape), _wspec(c1w.shape), _wspec(c2x2.shape),
                _wspec(bnn.shape)],
      out_specs=[_vspec(bn, hg), _vspec(bn, hg)],
      out_shape=[jax.ShapeDtypeStruct((n, hg), hn_dtype),
                 jax.ShapeDtypeStruct((n, hg), jnp.bfloat16)],
  )(cx, h, z, hr, lapn, p2n, dinv, c0m2, c1w, c2x2, bnn)


def _k_decoder(hs, hd, tc16, st, w_s, w_d, w_tblk, w_st, b_ih, w_hh, b_hh,
               w1, b1, w2, b2, e, t_steps, hg, hdec, be):
  hp = hs.shape[1]
  fs = w_st.shape[0]
  ftt = tc16.shape[1]

  def body(hs_ref, hd_ref, tc_ref, st_ref, ws_ref, wd_ref, wt_ref, wst_ref,
           bih_ref, whh_ref, bhh_ref, w1_ref, b1_ref, w2_ref, b2_ref, o_ref):
    gi_base = jnp.dot(hs_ref[...], ws_ref[...],
                      preferred_element_type=jnp.float32)
    gi_base += jnp.dot(hd_ref[...], wd_ref[...],
                       preferred_element_type=jnp.float32)
    gi_base += jnp.dot(st_ref[...], wst_ref[...],
                       preferred_element_type=jnp.float32)
    gi_base += bih_ref[...]
    gt_all = jnp.dot(tc_ref[...], wt_ref[...],
                     preferred_element_type=jnp.float32)
    h = jnp.zeros((be, hdec), jnp.float32)
    cols = []
    for t in range(t_steps):
      gi = gi_base + gt_all[:, 3 * hdec * t:3 * hdec * (t + 1)]
      gh = jnp.dot(h, whh_ref[...], preferred_element_type=jnp.float32)
      gh += bhh_ref[...]
      r = jax.nn.sigmoid(gi[:, :hdec] + gh[:, :hdec])
      z = jax.nn.sigmoid(gi[:, hdec:2 * hdec] + gh[:, hdec:2 * hdec])
      nn = jnp.tanh(gi[:, 2 * hdec:] + r * gh[:, 2 * hdec:])
      h = (1.0 - z) * nn + z * h
      hid = jax.nn.relu(
          jnp.dot(h, w1_ref[...], preferred_element_type=jnp.float32)
          + b1_ref[...])
      cols.append(jnp.dot(hid, w2_ref[...], preferred_element_type=jnp.float32)
                  + b2_ref[...])
    o_ref[...] = jnp.concatenate(cols, axis=1)

  return pl.pallas_call(
      body,
      grid=(e // be,),
      in_specs=[_vspec(be, hp), _vspec(be, hp), _vspec(be, ftt),
                _vspec(be, fs),
                _wspec(w_s.shape), _wspec(w_d.shape), _wspec(w_tblk.shape),
                _wspec(w_st.shape), _wspec(b_ih.shape), _wspec(w_hh.shape),
                _wspec(b_hh.shape), _wspec(w1.shape), _wspec(b1.shape),
                _wspec(w2.shape), _wspec(b2.shape)],
      out_specs=_vspec(be, t_steps),
      out_shape=jax.ShapeDtypeStruct((e, t_steps), jnp.float32),
  )(hs, hd, tc16, st, w_s, w_d, w_tblk, w_st, b_ih, w_hh, b_hh, w1, b1, w2, b2)


def kernel(x_seq, edge_index, time_seq, static_feats,
           W_xz, b_xz, W_hz, b_hz, W_xr, b_xr, W_hr, b_hr,
           W_xh, b_xh, W_hn, b_hn,
           W_ih, W_hh, b_ih, b_hh, W1, b1, W2, b2):
  t_steps, n, f_node = x_seq.shape
  e = edge_index.shape[1]
  hg = W_hz.shape[2]
  hdec = W_hh.shape[0] // 3
  ft = time_seq.shape[2]

  cpw = -(-e // (_NW * _CB))
  e_pad = _NW * cpw * _CB
  pad = e_pad - e
  bn = 2000
  be = 2000

  src = edge_index[0]
  dst = edge_index[1]
  zpad_i = jnp.zeros((pad,), jnp.int32)
  npad_i = jnp.full((pad,), n, jnp.int32)
  src_g = jnp.concatenate([src, zpad_i]).reshape(-1, _CB)
  dst_g = jnp.concatenate([dst, zpad_i]).reshape(-1, _CB)
  src_s = jnp.concatenate([src, npad_i]).reshape(-1, _CB)
  dst_s = jnp.concatenate([dst, npad_i]).reshape(-1, _CB)

  rpw = -(-(n + 1) // (_NS * 8)) * 8
  z_fn = jnp.zeros((rpw, f_node), jnp.bfloat16)
  z_hg = jnp.zeros((rpw, hg), jnp.bfloat16)
  ones_row = jnp.ones((_CB, 16), jnp.float32)

  seg_fn = _segsum(n, f_node, e_pad, tc_tiling=False, dtype=jnp.bfloat16,
                   nb=4)
  seg_hg = _segsum(n, hg, e_pad, tc_tiling=False,
                   dtype=jnp.bfloat16, nb=4)
  deg_k = _degree(n, 16, e_pad)
  gat_k = _gather2(n, hg, e_pad, tc_tiling=False,
                   dtype=jnp.bfloat16)

  degp = deg_k(src_s, ones_row, jnp.zeros((rpw, 16), jnp.float32))
  dinv, xs = _k_prescale(degp, x_seq, bn)

  a_k = [jnp.concatenate([W_xz[k], W_xr[k], W_xh[k]], axis=1) for k in range(3)]
  a0m2, a1w, a2x2 = a_k[0] - a_k[2], a_k[1], 2.0 * a_k[2]
  bx = jnp.concatenate([b_xz, b_xr, b_xh]).reshape(1, 3 * hg)
  b_k = [jnp.concatenate([W_hz[k], W_hr[k]], axis=1) for k in range(3)]
  b0m2, b1w, b2x2 = b_k[0] - b_k[2], b_k[1], 2.0 * b_k[2]
  bzr = jnp.concatenate([b_hz, b_hr]).reshape(1, 2 * hg)
  c0m2, c1w, c2x2 = W_hn[0] - W_hn[2], W_hn[1], 2.0 * W_hn[2]
  bz2 = b_hz.reshape(1, hg)
  bn2 = b_hn.reshape(1, hg)

  p1s = [seg_fn(xs[t], src_g, dst_s, z_fn) for t in range(t_steps)]
  mids = [_k_mid(p1, dinv, bn, m1_dtype=jnp.bfloat16) for p1 in p1s]
  p2s = [seg_fn(m1x, src_g, dst_s, z_fn) for _, m1x in mids]
  cxs = [_k_cheb(x_seq, mids[t][0], p2s[t], dinv, a0m2, a1w, a2x2, bx, bn, t)
         for t in range(t_steps)]

  h_cur, u_cur = _k_t0(cxs[0], dinv, bz2, bn2, hg, bn)
  for t in range(1, t_steps):
    p1h = seg_hg(u_cur, src_g, dst_s, z_hg)
    laph, m1h = _k_mid(p1h, dinv, bn, m1_dtype=jnp.bfloat16)
    p2h = seg_hg(m1h, src_g, dst_s, z_hg)
    z_gate, hr, uhr = _k_zr(cxs[t], h_cur, laph, p2h, dinv, b0m2, b1w, b2x2,
                            bzr, hg, bn)
    p1n = seg_hg(uhr, src_g, dst_s, z_hg)
    lapn, m1n = _k_mid(p1n, dinv, bn, m1_dtype=jnp.bfloat16)
    p2n = seg_hg(m1n, src_g, dst_s, z_hg)
    h_cur, u_cur = _k_upd(cxs[t], h_cur, z_gate, hr, lapn, p2n, dinv, c0m2,
                          c1w, c2x2, bn2, hg, bn,
                          hn_dtype=(jnp.bfloat16 if t == t_steps - 1
                                    else jnp.float32))

  hs, hd = gat_k(h_cur, src_g, dst_g)
  tc16 = jnp.transpose(time_seq, (1, 0, 2)).reshape(e, t_steps * ft)
  w_s = W_ih[:, :hg].T
  w_d = W_ih[:, hg:2 * hg].T
  w_t = W_ih[:, 2 * hg:2 * hg + ft].T
  w_st = W_ih[:, 2 * hg + ft:].T
  w_tblk = jnp.zeros((t_steps * ft, t_steps * 3 * hdec), jnp.float32)
  for t in range(t_steps):
    w_tblk = w_tblk.at[ft * t:ft * (t + 1),
                       3 * hdec * t:3 * hdec * (t + 1)].set(w_t)
  out_et = _k_decoder(hs, hd, tc16, static_feats,
                      w_s, w_d, w_tblk, w_st, b_ih.reshape(1, -1),
                      W_hh.T, b_hh.reshape(1, -1),
                      W1, b1.reshape(1, -1), W2, b2.reshape(1, -1),
                      e, t_steps, hg, hdec, be)
  return out_et.T
